# Initial kernel scaffold; baseline (speedup 1.0000x reference)
import jax
import jax.numpy as jnp
from jax import lax
from jax.experimental import pallas as pl
from jax.experimental.pallas import tpu as pltpu
from jax.experimental.pallas import tpu_sc as plsc

E = 1600000
G = 1024
NW = 32
CHUNK = E // NW
BLK = 2000
NBLK = CHUNK // BLK
VPB = BLK // 16
EPS = 1e-6
LN2 = 0.6931471805599453
LARGE_NEG = -1000000000.0


def _plog(x):
    bits = plsc.bitcast(x, jnp.int32)
    e = (bits >> 23) - 127
    mbits = (bits & 0x007FFFFF) | 0x3F800000
    m = plsc.bitcast(mbits, jnp.float32)
    big = m > 1.4142135623730951
    m = jnp.where(big, m * 0.5, m)
    e = jnp.where(big, e + 1, e)
    z = (m - 1.0) / (m + 1.0)
    z2 = z * z
    p = 2.0 * z * (1.0 + z2 * (1.0 / 3.0 + z2 * (1.0 / 5.0 + z2 * (1.0 / 7.0 + z2 * (1.0 / 9.0)))))
    return e.astype(jnp.float32) * LN2 + p


def _plogaddexp(a, b):
    m = jnp.maximum(a, b)
    d = jnp.minimum(a, b) - m
    return m + _plog(1.0 + jnp.exp(d))


def _phase1_body(s_hbm, r_hbm, u_hbm, b_hbm,
                 cnt_out, s1_out, s2_out, w_out, arg_out,
                 sbuf, rbuf, ubuf, bbuf,
                 cnt_t, s1_t, s2_t, w_t, arg_t,
                 tmp_sc, tmp_e1, tmp_w,
                 sem0, sem1):
    info = plsc.get_sparse_core_info()
    ns = info.num_subcores
    wid = lax.axis_index("c") * ns + lax.axis_index("s")
    base = wid * CHUNK

    zf16 = jnp.zeros((16,), jnp.float32)
    zi16 = jnp.zeros((16,), jnp.int32)
    eidx16 = jnp.full((16,), E, jnp.int32)
    iota16 = lax.iota(jnp.int32, 16)
    l0m = iota16 == 0

    def init_step(k, _):
        sl = pl.ds(k * 16, 16)
        cnt_t[sl] = zi16
        s1_t[sl] = zf16
        s2_t[sl] = zf16
        w_t[sl] = zf16
        arg_t[sl] = eidx16
        return 0
    lax.fori_loop(0, G // 16, init_step, 0)

    hbufs = (s_hbm, r_hbm, u_hbm, b_hbm)
    vbufs = (sbuf, rbuf, ubuf, bbuf)
    sems = (sem0, sem1)

    def issue(blk, slot):
        for h, v in zip(hbufs, vbufs):
            pltpu.async_copy(h.at[pl.ds(base + blk * BLK, BLK)],
                             v.at[pl.ds(slot * BLK, BLK)], sems[slot])

    def drain(blk, slot):
        for h, v in zip(hbufs, vbufs):
            pltpu.make_async_copy(h.at[pl.ds(base + blk * BLK, BLK)],
                                  v.at[pl.ds(slot * BLK, BLK)], sems[slot]).wait()

    def flush(cur_g, run_start, p, s1v, s2v, wv, argv):
        cnt_t[cur_g] = p - run_start
        s1_t[cur_g] = jnp.sum(s1v)
        s2_t[cur_g] = jnp.sum(s2v)
        m = jnp.max(wv)
        w_t[cur_g] = m
        arg_t[cur_g] = jnp.min(jnp.where(wv == m, argv, eidx16))

    issue(0, 0)
    carry = (jnp.int32(-1), jnp.int32(0), zf16, zf16, zf16, eidx16)

    for blk in range(NBLK):
        slot = blk % 2
        if blk + 1 < NBLK:
            issue(blk + 1, 1 - slot)
        drain(blk, slot)

        def vec_step(j, c, blk=blk, slot=slot):
            cur_g, run_start, s1v, s2v, wvacc, argv = c
            off = slot * BLK + j * 16
            p0 = base + blk * BLK + j * 16
            b16 = bbuf[pl.ds(off, 16)]
            s16 = sbuf[pl.ds(off, 16)]
            r16 = rbuf[pl.ds(off, 16)]
            u16 = ubuf[pl.ds(off, 16)]
            scv = jnp.maximum(s16, EPS)
            e1v = scv * jnp.exp(r16)
            wv = e1v / (-_plog(u16))
            common = jnp.all(b16 == cur_g)

            def common_fn(c):
                cur_g, run_start, s1v, s2v, wvacc, argv = c
                upd = wv > wvacc
                return (cur_g, run_start, s1v + scv, s2v + e1v,
                        jnp.where(upd, wv, wvacc),
                        jnp.where(upd, p0 + iota16, argv))

            def boundary_fn(c):
                tmp_sc[...] = scv
                tmp_e1[...] = e1v
                tmp_w[...] = wv

                def elem(i, cc):
                    cur_g, run_start, s1v, s2v, wvacc, argv = cc
                    b_i = bbuf[off + i]
                    p_i = p0 + i
                    change = b_i != cur_g

                    @pl.when(change & (cur_g >= 0))
                    def _():
                        flush(cur_g, run_start, p_i, s1v, s2v, wvacc, argv)

                    run_start = jnp.where(change, p_i, run_start)
                    cur_g = jnp.where(change, b_i, cur_g)
                    s1v = jnp.where(change, zf16, s1v)
                    s2v = jnp.where(change, zf16, s2v)
                    wvacc = jnp.where(change, zf16, wvacc)
                    argv = jnp.where(change, eidx16, argv)

                    sc_s = tmp_sc[i]
                    e1_s = tmp_e1[i]
                    w_s = tmp_w[i]
                    s1v = s1v + jnp.where(l0m, sc_s, 0.0)
                    s2v = s2v + jnp.where(l0m, e1_s, 0.0)
                    upd = l0m & (w_s > wvacc)
                    wvacc = jnp.where(upd, w_s, wvacc)
                    argv = jnp.where(upd, p_i, argv)
                    return (cur_g, run_start, s1v, s2v, wvacc, argv)

                return lax.fori_loop(0, 16, elem, c)

            return lax.cond(common, common_fn, boundary_fn, c)

        carry = lax.fori_loop(0, VPB, vec_step, carry)

    cur_g, run_start, s1v, s2v, wvacc, argv = carry

    @pl.when(cur_g >= 0)
    def _():
        flush(cur_g, run_start, jnp.int32(base + CHUNK), s1v, s2v, wvacc, argv)

    pltpu.sync_copy(cnt_t, cnt_out.at[wid])
    pltpu.sync_copy(s1_t, s1_out.at[wid])
    pltpu.sync_copy(s2_t, s2_out.at[wid])
    pltpu.sync_copy(w_t, w_out.at[wid])
    pltpu.sync_copy(arg_t, arg_out.at[wid])


def _phase2_body(cnt_hbm, s1_hbm, s2_hbm, w_hbm, arg_hbm,
                 sr_hbm, us_hbm, s_hbm, r_hbm,
                 act_out, pf_out,
                 cntv, s1b, s2b, wb, ab, totc, ptb,
                 srb, usb, idxb, gsb, grb,
                 lse12b, stopb, chooseb, actb, pfb,
                 sem):
    info = plsc.get_sparse_core_info()
    ns = info.num_subcores
    wid = lax.axis_index("c") * ns + lax.axis_index("s")
    gbase = wid * 32

    pltpu.async_copy(cnt_hbm, cntv, sem)
    for src in range(NW):
        pltpu.async_copy(s1_hbm.at[src, pl.ds(gbase, 32)], s1b.at[pl.ds(src * 32, 32)], sem)
        pltpu.async_copy(s2_hbm.at[src, pl.ds(gbase, 32)], s2b.at[pl.ds(src * 32, 32)], sem)
        pltpu.async_copy(w_hbm.at[src, pl.ds(gbase, 32)], wb.at[pl.ds(src * 32, 32)], sem)
        pltpu.async_copy(arg_hbm.at[src, pl.ds(gbase, 32)], ab.at[pl.ds(src * 32, 32)], sem)
    pltpu.async_copy(sr_hbm.at[pl.ds(gbase, 32)], srb, sem)
    pltpu.async_copy(us_hbm.at[pl.ds(gbase, 32)], usb, sem)
    pltpu.make_async_copy(cnt_hbm, cntv, sem).wait()
    for src in range(NW):
        pltpu.make_async_copy(s1_hbm.at[src, pl.ds(gbase, 32)], s1b.at[pl.ds(src * 32, 32)], sem).wait()
        pltpu.make_async_copy(s2_hbm.at[src, pl.ds(gbase, 32)], s2b.at[pl.ds(src * 32, 32)], sem).wait()
        pltpu.make_async_copy(w_hbm.at[src, pl.ds(gbase, 32)], wb.at[pl.ds(src * 32, 32)], sem).wait()
        pltpu.make_async_copy(arg_hbm.at[src, pl.ds(gbase, 32)], ab.at[pl.ds(src * 32, 32)], sem).wait()
    pltpu.make_async_copy(sr_hbm.at[pl.ds(gbase, 32)], srb, sem).wait()
    pltpu.make_async_copy(us_hbm.at[pl.ds(gbase, 32)], usb, sem).wait()

    def tot_step(k, _):
        def src_step(src, acc):
            return acc + cntv[pl.ds(src * G + k * 16, 16)]
        totc[pl.ds(k * 16, 16)] = lax.fori_loop(0, NW, src_step, jnp.zeros((16,), jnp.int32))
        return 0
    lax.fori_loop(0, G // 16, tot_step, 0)

    def cum_step(k, cry):
        v = totc[pl.ds(k * 16, 16)]
        ptb[pl.ds(k * 16, 16)] = plsc.cumsum(v) + cry
        return cry + jnp.sum(v)
    lax.fori_loop(0, G // 16, cum_step, jnp.int32(0))

    for h in range(2):
        gsl = pl.ds(gbase + h * 16, 16)
        n16 = totc[gsl]

        def merge_step(src, acc, h=h):
            s1a, s2a, wm, am = acc
            sl = pl.ds(src * 32 + h * 16, 16)
            wv = wb[sl]
            upd = wv > wm
            return (s1a + s1b[sl], s2a + s2b[sl],
                    jnp.where(upd, wv, wm), jnp.where(upd, ab[sl], am))
        s1m, s2m, wm, am = lax.fori_loop(
            0, NW, merge_step,
            (jnp.zeros((16,), jnp.float32), jnp.zeros((16,), jnp.float32),
             jnp.zeros((16,), jnp.float32), jnp.full((16,), E, jnp.int32)))

        has = n16 > 0
        nf = jnp.maximum(n16.astype(jnp.float32), 1.0)
        ls1 = -_plog(nf)
        lse1 = jnp.where(has, _plogaddexp(_plog(s1m), ls1), ls1)
        sr16 = srb[pl.ds(h * 16, 16)]
        us16 = usb[pl.ds(h * 16, 16)]
        ls2 = ls1 - lse1 + sr16
        lse2 = jnp.where(has, _plogaddexp(_plog(s2m) - lse1, ls2), ls2)
        semax = jnp.where(has, _plog(wm) - lse1 - lse2, LARGE_NEG)
        gstop = -_plog(-_plog(us16))
        sstop = ls2 - lse2 + gstop
        choose = has & (semax > sstop)
        argc = jnp.where(has, jnp.clip(am, 0, E - 1), 0)

        hsl = pl.ds(h * 16, 16)
        idxb[hsl] = argc
        lse12b[hsl] = lse1 + lse2
        stopb[hsl] = ls2 - lse2
        chooseb[hsl] = choose.astype(jnp.int32)
        actb[hsl] = jnp.where(choose, argc, ptb[gsl])

    pltpu.async_copy(s_hbm.at[idxb], gsb, sem)
    pltpu.async_copy(r_hbm.at[idxb], grb, sem)
    pltpu.make_async_copy(s_hbm.at[idxb], gsb, sem).wait()
    pltpu.make_async_copy(r_hbm.at[idxb], grb, sem).wait()

    for h in range(2):
        hsl = pl.ds(h * 16, 16)
        lr = _plog(jnp.maximum(gsb[hsl], EPS)) + grb[hsl]
        choose = chooseb[hsl] != 0
        pfb[hsl] = jnp.where(choose, lr - lse12b[hsl], stopb[hsl])

    pltpu.sync_copy(actb, act_out.at[pl.ds(gbase, 32)])
    pltpu.sync_copy(pfb, pf_out.at[pl.ds(gbase, 32)])


def kernel(edge_scores, edge_residual, stop_residual, noise_edge_u, noise_stop_u, edge_batch):
    mesh = plsc.VectorSubcoreMesh(core_axis_name="c", subcore_axis_name="s")
    f32 = jnp.float32
    i32 = jnp.int32

    phase1 = pl.kernel(
        _phase1_body,
        out_type=(
            jax.ShapeDtypeStruct((NW, G), i32),
            jax.ShapeDtypeStruct((NW, G), f32),
            jax.ShapeDtypeStruct((NW, G), f32),
            jax.ShapeDtypeStruct((NW, G), f32),
            jax.ShapeDtypeStruct((NW, G), i32),
        ),
        mesh=mesh,
        scratch_types=[
            pltpu.VMEM((2 * BLK,), f32),
            pltpu.VMEM((2 * BLK,), f32),
            pltpu.VMEM((2 * BLK,), f32),
            pltpu.VMEM((2 * BLK,), i32),
            pltpu.VMEM((G,), i32),
            pltpu.VMEM((G,), f32),
            pltpu.VMEM((G,), f32),
            pltpu.VMEM((G,), f32),
            pltpu.VMEM((G,), i32),
            pltpu.VMEM((16,), f32),
            pltpu.VMEM((16,), f32),
            pltpu.VMEM((16,), f32),
            pltpu.SemaphoreType.DMA,
            pltpu.SemaphoreType.DMA,
        ],
    )
    cnt, s1, s2, wm, ag = phase1(edge_scores, edge_residual, noise_edge_u, edge_batch)

    phase2 = pl.kernel(
        _phase2_body,
        out_type=(
            jax.ShapeDtypeStruct((G,), i32),
            jax.ShapeDtypeStruct((G,), f32),
        ),
        mesh=mesh,
        scratch_types=[
            pltpu.VMEM((NW * G,), i32),
            pltpu.VMEM((NW * 32,), f32),
            pltpu.VMEM((NW * 32,), f32),
            pltpu.VMEM((NW * 32,), f32),
            pltpu.VMEM((NW * 32,), i32),
            pltpu.VMEM((G,), i32),
            pltpu.VMEM((G,), i32),
            pltpu.VMEM((32,), f32),
            pltpu.VMEM((32,), f32),
            pltpu.VMEM((32,), i32),
            pltpu.VMEM((32,), f32),
            pltpu.VMEM((32,), f32),
            pltpu.VMEM((32,), f32),
            pltpu.VMEM((32,), f32),
            pltpu.VMEM((32,), i32),
            pltpu.VMEM((32,), i32),
            pltpu.VMEM((32,), f32),
            pltpu.SemaphoreType.DMA,
        ],
    )
    cnt_flat = cnt.reshape((NW * G,))
    actions, log_pf = phase2(cnt_flat, s1, s2, wm, ag,
                             stop_residual, noise_stop_u,
                             edge_scores, edge_residual)
    return actions, log_pf


# speedup vs baseline: 239.6051x; 239.6051x over previous
import jax
import jax.numpy as jnp
from jax import lax
from jax.experimental import pallas as pl
from jax.experimental.pallas import tpu as pltpu
from jax.experimental.pallas import tpu_sc as plsc

E = 1600000
G = 1024
NW = 32
CHUNK = E // NW
BLK = 2000
NBLK = CHUNK // BLK
VPB = BLK // 16
EPS = 1e-6
LN2 = 0.6931471805599453
LARGE_NEG = -1000000000.0
EF = float(E)

f32 = jnp.float32
i32 = jnp.int32


def _iota():
    return lax.iota(i32, 16)


def _plog(x):
    bits = lax.bitcast_convert_type(x, i32)
    e = (bits >> 23) - 127
    mbits = (bits & 0x007FFFFF) | 0x3F800000
    m = lax.bitcast_convert_type(mbits, f32)
    big = m > 1.4142135623730951
    m = jnp.where(big, m * 0.5, m)
    e = jnp.where(big, e + 1, e)
    z = (m - 1.0) / (m + 1.0)
    z2 = z * z
    p = 2.0 * z * (1.0 + z2 * (1.0 / 3.0 + z2 * (1.0 / 5.0 + z2 * (1.0 / 7.0 + z2 * (1.0 / 9.0)))))
    return e.astype(f32) * LN2 + p


def _plogaddexp(a, b):
    m = jnp.maximum(a, b)
    d = jnp.minimum(a, b) - m
    return m + _plog(1.0 + jnp.exp(d))


def _bf_sum(v):
    it = _iota()
    for k in (1, 2, 4, 8):
        v = v + v[it ^ k]
    return v


def _bf_maxarg(w, a, e1):
    it = _iota()
    for k in (1, 2, 4, 8):
        wp = w[it ^ k]
        ap = a[it ^ k]
        ep = e1[it ^ k]
        take = (wp > w) | ((wp == w) & (ap < a))
        w = jnp.where(take, wp, w)
        a = jnp.where(take, ap, a)
        e1 = jnp.where(take, ep, e1)
    return w, a, e1


def _phase1_body(s_hbm, r_hbm, u_hbm, b_hbm,
                 cnt_out, pk_out,
                 sbuf, rbuf, ubuf, bbuf,
                 packed, cntc,
                 cgv, rsv, as1, as2, aw, aarg, ae1, tmpv,
                 sem0, sem1):
    info = plsc.get_sparse_core_info()
    ns = info.num_subcores
    wid = lax.axis_index("c") * ns + lax.axis_index("s")
    base = wid * CHUNK

    it = _iota()
    l0m = it == 0
    all16 = pl.ds(0, 16)
    zf = jnp.zeros((16,), f32)

    init_row = jnp.where(it == 4, EF, jnp.where(it == 5, 1.0, 0.0))

    def init_step(g, _):
        packed[pl.ds(g * 16, 16)] = init_row
        return 0
    lax.fori_loop(0, G, init_step, 0)

    cgv[all16] = jnp.full((16,), -1, i32)
    rsv[all16] = jnp.zeros((16,), i32)
    as1[all16] = zf
    as2[all16] = zf
    aw[all16] = zf
    aarg[all16] = jnp.full((16,), EF, f32)
    ae1[all16] = jnp.full((16,), 1.0, f32)

    hbufs = (s_hbm, r_hbm, u_hbm, b_hbm)
    vbufs = (sbuf, rbuf, ubuf, bbuf)

    def issue(blk, slot_off, sem):
        for h, v in zip(hbufs, vbufs):
            pltpu.async_copy(h.at[pl.ds(base + blk * BLK, BLK)],
                             v.at[pl.ds(slot_off, BLK)], sem)

    def drain(blk, slot_off, sem):
        for h, v in zip(hbufs, vbufs):
            pltpu.make_async_copy(h.at[pl.ds(base + blk * BLK, BLK)],
                                  v.at[pl.ds(slot_off, BLK)], sem).wait()

    def flush(cg, p):
        rs = rsv[all16][0]
        cntf = (p - rs).astype(f32)
        s1 = _bf_sum(as1[all16])
        s2 = _bf_sum(as2[all16])
        wr, ar, er = _bf_maxarg(aw[all16], aarg[all16], ae1[all16])
        row = jnp.where(it == 0, cntf, 0.0)
        row = jnp.where(it == 1, s1, row)
        row = jnp.where(it == 2, s2, row)
        row = jnp.where(it == 3, wr, row)
        row = jnp.where(it == 4, ar, row)
        row = jnp.where(it == 5, er, row)
        packed[pl.ds(cg * 16, 16)] = row

    issue(0, 0, sem0)

    def block_step(blk, _):
        slot = lax.rem(blk, 2)
        soff = slot * BLK

        @pl.when((blk + 1 < NBLK) & (slot == 0))
        def _():
            issue(blk + 1, BLK, sem1)

        @pl.when((blk + 1 < NBLK) & (slot == 1))
        def _():
            issue(blk + 1, 0, sem0)

        @pl.when(slot == 0)
        def _():
            drain(blk, 0, sem0)

        @pl.when(slot == 1)
        def _():
            drain(blk, BLK, sem1)

        def vec_step(j, _):
            off = soff + j * 16
            p0 = base + blk * BLK + j * 16
            b16 = bbuf[pl.ds(off, 16)]
            s16 = sbuf[pl.ds(off, 16)]
            r16 = rbuf[pl.ds(off, 16)]
            u16 = ubuf[pl.ds(off, 16)]
            scv = jnp.maximum(s16, EPS)
            e1v = scv * jnp.exp(r16)
            wv = e1v / (-_plog(u16))
            cg = cgv[all16][0]
            common = (b16[0] == cg) & (b16[15] == cg)

            @pl.when(common)
            def _():
                as1[all16] = as1[all16] + scv
                as2[all16] = as2[all16] + e1v
                awr = aw[all16]
                up = wv > awr
                aw[all16] = jnp.where(up, wv, awr)
                posf = (p0 + it).astype(f32)
                aarg[all16] = jnp.where(up, posf, aarg[all16])
                ae1[all16] = jnp.where(up, e1v, ae1[all16])

            @pl.when(jnp.logical_not(common))
            def _():
                b16f = b16.astype(f32)

                def elem(i, _):
                    isel = (it & 0) + i
                    brep = b16f[isel]
                    screp = scv[isel]
                    e1rep = e1v[isel]
                    wrep = wv[isel]
                    tmpv[all16] = brep
                    b_i = tmpv[all16][0].astype(i32)
                    p_i = p0 + i
                    cg_i = cgv[all16][0]
                    change = b_i != cg_i

                    @pl.when(change & (cg_i >= 0))
                    def _():
                        flush(cg_i, p_i)

                    @pl.when(change)
                    def _():
                        cgv[all16] = jnp.full((16,), b_i, i32)
                        rsv[all16] = jnp.full((16,), p_i, i32)
                        as1[all16] = zf
                        as2[all16] = zf
                        aw[all16] = zf
                        aarg[all16] = jnp.full((16,), EF, f32)
                        ae1[all16] = jnp.full((16,), 1.0, f32)

                    as1[all16] = as1[all16] + jnp.where(l0m, screp, 0.0)
                    as2[all16] = as2[all16] + jnp.where(l0m, e1rep, 0.0)
                    awr = aw[all16]
                    up = l0m & (wrep > awr)
                    aw[all16] = jnp.where(up, wrep, awr)
                    aarg[all16] = jnp.where(up, jnp.full((16,), p_i, i32).astype(f32), aarg[all16])
                    ae1[all16] = jnp.where(up, e1rep, ae1[all16])
                    return 0

                lax.fori_loop(0, 16, elem, 0)

            return 0

        lax.fori_loop(0, VPB, vec_step, 0)
        return 0

    lax.fori_loop(0, NBLK, block_step, 0)

    cg_f = cgv[all16][0]

    @pl.when(cg_f >= 0)
    def _():
        flush(cg_f, jnp.int32(base + CHUNK))

    def cstep(go, _):
        def cinner(j, acc):
            row = packed[pl.ds((go * 16 + j) * 16, 16)]
            return jnp.where(it == j, row[0], acc)
        cntc[pl.ds(go * 16, 16)] = lax.fori_loop(0, 16, cinner, zf)
        return 0
    lax.fori_loop(0, G // 16, cstep, 0)

    pltpu.sync_copy(cntc, cnt_out.at[wid])
    pltpu.sync_copy(packed, pk_out.at[wid])


def _prefix16(v):
    it = _iota()
    for k in (1, 2, 4, 8):
        v = v + jnp.where(it >= k, v[jnp.maximum(it - k, 0)], 0.0)
    return v


def _phase2_body(cnt_hbm, pk_hbm, sr_hbm, us_hbm,
                 act_out, pf_out,
                 cntv, pbuf, totc, ptb,
                 srb, usb, actb, pfb,
                 sem):
    info = plsc.get_sparse_core_info()
    ns = info.num_subcores
    wid = lax.axis_index("c") * ns + lax.axis_index("s")
    gbase = wid * 32
    it = _iota()
    all16 = pl.ds(0, 16)

    pltpu.async_copy(cnt_hbm, cntv, sem)
    for src in range(NW):
        pltpu.async_copy(pk_hbm.at[src, pl.ds(gbase * 16, 512)],
                         pbuf.at[pl.ds(src * 512, 512)], sem)
    pltpu.async_copy(sr_hbm.at[pl.ds(gbase, 32)], srb, sem)
    pltpu.async_copy(us_hbm.at[pl.ds(gbase, 32)], usb, sem)
    pltpu.make_async_copy(cnt_hbm, cntv, sem).wait()
    for src in range(NW):
        pltpu.make_async_copy(pk_hbm.at[src, pl.ds(gbase * 16, 512)],
                              pbuf.at[pl.ds(src * 512, 512)], sem).wait()
    pltpu.make_async_copy(sr_hbm.at[pl.ds(gbase, 32)], srb, sem).wait()
    pltpu.make_async_copy(us_hbm.at[pl.ds(gbase, 32)], usb, sem).wait()

    def tot_step(k, _):
        def src_step(src, acc):
            return acc + cntv[pl.ds(src * G + k * 16, 16)]
        totc[pl.ds(k * 16, 16)] = lax.fori_loop(0, NW, src_step, jnp.zeros((16,), f32))
        return 0
    lax.fori_loop(0, G // 16, tot_step, 0)

    def cum_step(k, cry):
        v = _prefix16(totc[pl.ds(k * 16, 16)]) + cry
        ptb[pl.ds(k * 16, 16)] = v
        return v[15]
    lax.fori_loop(0, G // 16, cum_step, jnp.float32(0))

    sum_lanes = it <= 2

    for h in range(2):
        n16 = jnp.zeros((16,), f32)
        s1m = jnp.zeros((16,), f32)
        s2m = jnp.zeros((16,), f32)
        wm = jnp.zeros((16,), f32)
        am = jnp.zeros((16,), f32)
        e1m = jnp.zeros((16,), f32)
        for j in range(16):
            glocal = h * 16 + j

            def src_step(src, acc, glocal=glocal):
                row = pbuf[pl.ds(src * 512 + glocal * 16, 16)]
                w_p = row[3]
                w_a = acc[3]
                take = (w_p > w_a) | ((w_p == w_a) & (row[4] < acc[4]))
                return jnp.where(sum_lanes, acc + row, jnp.where(take, row, acc))
            acc = lax.fori_loop(0, NW, src_step,
                                jnp.where(it == 4, EF, jnp.where(it == 5, 1.0, 0.0)))
            lane = it == j
            n16 = jnp.where(lane, acc[0], n16)
            s1m = jnp.where(lane, acc[1], s1m)
            s2m = jnp.where(lane, acc[2], s2m)
            wm = jnp.where(lane, acc[3], wm)
            am = jnp.where(lane, acc[4], am)
            e1m = jnp.where(lane, acc[5], e1m)

        has = n16 > 0
        nf = jnp.maximum(n16, 1.0)
        ls1 = -_plog(nf)
        s1safe = jnp.maximum(s1m, 1e-30)
        s2safe = jnp.maximum(s2m, 1e-30)
        lse1 = jnp.where(has, _plogaddexp(_plog(s1safe), ls1), ls1)
        sr16 = srb[pl.ds(h * 16, 16)]
        us16 = usb[pl.ds(h * 16, 16)]
        ls2 = ls1 - lse1 + sr16
        lse2 = jnp.where(has, _plogaddexp(_plog(s2safe) - lse1, ls2), ls2)
        semax = jnp.where(has, _plog(jnp.maximum(wm, 1e-30)) - lse1 - lse2, LARGE_NEG)
        gstop = -_plog(-_plog(us16))
        sstop = ls2 - lse2 + gstop
        choose = has & (semax > sstop)
        argc = jnp.where(has, jnp.clip(am, 0.0, EF - 1.0), 0.0)
        lr = _plog(jnp.maximum(e1m, 1e-30))
        ptr16 = ptb[pl.ds(gbase + h * 16, 16)]
        act = jnp.where(choose, argc, ptr16)
        pf = jnp.where(choose, lr - lse1 - lse2, ls2 - lse2)
        hsl = pl.ds(h * 16, 16)
        actb[hsl] = act.astype(i32)
        pfb[hsl] = pf

    pltpu.sync_copy(actb, act_out.at[pl.ds(gbase, 32)])
    pltpu.sync_copy(pfb, pf_out.at[pl.ds(gbase, 32)])


def kernel(edge_scores, edge_residual, stop_residual, noise_edge_u, noise_stop_u, edge_batch):
    mesh = plsc.VectorSubcoreMesh(core_axis_name="c", subcore_axis_name="s")

    phase1 = pl.kernel(
        _phase1_body,
        out_type=(
            jax.ShapeDtypeStruct((NW, G), f32),
            jax.ShapeDtypeStruct((NW, G * 16), f32),
        ),
        mesh=mesh,
        scratch_types=[
            pltpu.VMEM((2 * BLK,), f32),
            pltpu.VMEM((2 * BLK,), f32),
            pltpu.VMEM((2 * BLK,), f32),
            pltpu.VMEM((2 * BLK,), i32),
            pltpu.VMEM((G * 16,), f32),
            pltpu.VMEM((G,), f32),
            pltpu.VMEM((16,), i32),
            pltpu.VMEM((16,), i32),
            pltpu.VMEM((16,), f32),
            pltpu.VMEM((16,), f32),
            pltpu.VMEM((16,), f32),
            pltpu.VMEM((16,), f32),
            pltpu.VMEM((16,), f32),
            pltpu.VMEM((16,), f32),
            pltpu.SemaphoreType.DMA,
            pltpu.SemaphoreType.DMA,
        ],
    )
    cnt, pk = phase1(edge_scores, edge_residual, noise_edge_u, edge_batch)

    phase2 = pl.kernel(
        _phase2_body,
        out_type=(
            jax.ShapeDtypeStruct((G,), i32),
            jax.ShapeDtypeStruct((G,), f32),
        ),
        mesh=mesh,
        scratch_types=[
            pltpu.VMEM((NW * G,), f32),
            pltpu.VMEM((NW * 512,), f32),
            pltpu.VMEM((G,), f32),
            pltpu.VMEM((G,), f32),
            pltpu.VMEM((32,), f32),
            pltpu.VMEM((32,), f32),
            pltpu.VMEM((32,), i32),
            pltpu.VMEM((32,), f32),
            pltpu.SemaphoreType.DMA,
        ],
    )
    actions, log_pf = phase2(cnt.reshape((NW * G,)), pk,
                             stop_residual, noise_stop_u)
    return actions, log_pf


# speedup vs baseline: 450.8363x; 1.8816x over previous
import jax
import jax.numpy as jnp
from jax import lax
from jax.experimental import pallas as pl
from jax.experimental.pallas import tpu as pltpu
from jax.experimental.pallas import tpu_sc as plsc

E = 1600000
G = 1024
NW = 32
CHUNK = E // NW
BLK = 2000
NBLK = CHUNK // BLK
VPB = BLK // 16
EPS = 1e-6
LN2 = 0.6931471805599453
LARGE_NEG = -1000000000.0
EF = float(E)

f32 = jnp.float32
i32 = jnp.int32


def _iota():
    return lax.iota(i32, 16)


_LC = (2.0086062590962683e-08, 0.9999999403953552, -0.5000073909759521,
       0.33334827423095703, -0.24958817660808563, 0.19907750189304352,
       -0.17360951006412506, 0.16165275871753693, -0.09719803929328918)


def _plog(x):
    bits = lax.bitcast_convert_type(x, i32)
    e = (bits >> 23) - 127
    mbits = (bits & 0x007FFFFF) | 0x3F800000
    m = lax.bitcast_convert_type(mbits, f32)
    big = m > 1.4142135623730951
    m = jnp.where(big, m * 0.5, m)
    e = jnp.where(big, e + 1, e)
    t = m - 1.0
    t2 = t * t
    t4 = t2 * t2
    p = ((_LC[0] + _LC[1] * t) + t2 * (_LC[2] + _LC[3] * t)
         + t4 * ((_LC[4] + _LC[5] * t) + t2 * (_LC[6] + _LC[7] * t) + t4 * _LC[8]))
    return e.astype(f32) * LN2 + p


def _plogaddexp(a, b):
    m = jnp.maximum(a, b)
    d = jnp.minimum(a, b) - m
    return m + _plog(1.0 + jnp.exp(d))


def _bf_sum(v):
    it = _iota()
    for k in (1, 2, 4, 8):
        v = v + v[it ^ k]
    return v


def _bf_maxarg(w, a, e1):
    it = _iota()
    for k in (1, 2, 4, 8):
        wp = w[it ^ k]
        ap = a[it ^ k]
        ep = e1[it ^ k]
        take = (wp > w) | ((wp == w) & (ap < a))
        w = jnp.where(take, wp, w)
        a = jnp.where(take, ap, a)
        e1 = jnp.where(take, ep, e1)
    return w, a, e1


GRP = 5
GPB = VPB // GRP


def _phase1_body(s_hbm, r_hbm, u_hbm, b_hbm,
                 cnt_out, pk_out,
                 sbuf, rbuf, ubuf, bbuf,
                 packed, cntc,
                 cgv, rsv, as1, as2, anum, aden, apos, tmpv,
                 sem0, sem1):
    info = plsc.get_sparse_core_info()
    ns = info.num_subcores
    wid = lax.axis_index("c") * ns + lax.axis_index("s")
    base = wid * CHUNK

    it = _iota()
    l0m = it == 0
    all16 = pl.ds(0, 16)
    zf = jnp.zeros((16,), f32)
    onef = jnp.full((16,), 1.0, f32)
    efv = jnp.full((16,), EF, f32)

    init_row = jnp.where(it == 4, EF, jnp.where(it == 5, 1.0, 0.0))

    def init_step(g, _):
        packed[pl.ds(g * 16, 16)] = init_row
        return 0
    lax.fori_loop(0, G, init_step, 0)

    cgv[all16] = jnp.full((16,), -1, i32)
    rsv[all16] = jnp.zeros((16,), i32)

    hbufs = (s_hbm, r_hbm, u_hbm, b_hbm)
    vbufs = (sbuf, rbuf, ubuf, bbuf)

    def issue(blk, slot_off, sem):
        for h, v in zip(hbufs, vbufs):
            pltpu.async_copy(h.at[pl.ds(base + blk * BLK, BLK)],
                             v.at[pl.ds(slot_off, BLK)], sem)

    def drain(blk, slot_off, sem):
        for h, v in zip(hbufs, vbufs):
            pltpu.make_async_copy(h.at[pl.ds(base + blk * BLK, BLK)],
                                  v.at[pl.ds(slot_off, BLK)], sem).wait()

    def flush(cg, p):
        rs = rsv[all16][0]
        cntf = (p - rs).astype(f32)
        s1 = _bf_sum(as1[all16])
        s2 = _bf_sum(as2[all16])
        nr = anum[all16]
        wlane = nr / aden[all16]
        wr, ar, er = _bf_maxarg(wlane, apos[all16], nr)
        row = jnp.where(it == 0, cntf, 0.0)
        row = jnp.where(it == 1, s1, row)
        row = jnp.where(it == 2, s2, row)
        row = jnp.where(it == 3, wr, row)
        row = jnp.where(it == 4, ar, row)
        row = jnp.where(it == 5, er, row)
        packed[pl.ds(cg * 16, 16)] = row

    def slow_vec(v, off0, p00):
        off = off0 + v * 16
        p0 = p00 + v * 16
        b16 = bbuf[pl.ds(off, 16)]
        s16 = sbuf[pl.ds(off, 16)]
        r16 = rbuf[pl.ds(off, 16)]
        u16 = ubuf[pl.ds(off, 16)]
        scv = jnp.maximum(s16, EPS)
        e1v = scv * jnp.exp(r16)
        yv = -_plog(u16)
        cg = cgv[all16][0]
        common = (b16[0] == cg) & (b16[15] == cg)

        @pl.when(common)
        def _():
            as1[all16] = as1[all16] + scv
            as2[all16] = as2[all16] + e1v
            nr = anum[all16]
            dr = aden[all16]
            up = e1v * dr > nr * yv
            anum[all16] = jnp.where(up, e1v, nr)
            aden[all16] = jnp.where(up, yv, dr)
            posf = (p0 + it).astype(f32)
            apos[all16] = jnp.where(up, posf, apos[all16])

        @pl.when(jnp.logical_not(common))
        def _():
            b16f = b16.astype(f32)

            def elem(i, _):
                isel = (it & 0) + i
                brep = b16f[isel]
                screp = scv[isel]
                e1rep = e1v[isel]
                yrep = yv[isel]
                tmpv[all16] = brep
                b_i = tmpv[all16][0].astype(i32)
                p_i = p0 + i
                cg_i = cgv[all16][0]
                change = b_i != cg_i

                @pl.when(change & (cg_i >= 0))
                def _():
                    flush(cg_i, p_i)

                @pl.when(change)
                def _():
                    cgv[all16] = jnp.full((16,), b_i, i32)
                    rsv[all16] = jnp.full((16,), p_i, i32)
                    as1[all16] = zf
                    as2[all16] = zf
                    anum[all16] = zf
                    aden[all16] = onef
                    apos[all16] = efv

                as1[all16] = as1[all16] + jnp.where(l0m, screp, 0.0)
                as2[all16] = as2[all16] + jnp.where(l0m, e1rep, 0.0)
                nr = anum[all16]
                dr = aden[all16]
                up = l0m & (e1rep * dr > nr * yrep)
                anum[all16] = jnp.where(up, e1rep, nr)
                aden[all16] = jnp.where(up, yrep, dr)
                apos[all16] = jnp.where(up, jnp.full((16,), p_i, i32).astype(f32), apos[all16])
                return 0

            lax.fori_loop(0, 16, elem, 0)

    def comb(a, b):
        an, ad, ap = a
        bn, bd, bp = b
        take = bn * ad > an * bd
        return (jnp.where(take, bn, an), jnp.where(take, bd, ad),
                jnp.where(take, bp, ap))

    issue(0, 0, sem0)
    car0 = (zf, zf, zf, onef, efv, jnp.int32(-1))

    def block_step(blk, car):
        slot = lax.rem(blk, 2)
        soff = slot * BLK

        @pl.when((blk + 1 < NBLK) & (slot == 0))
        def _():
            issue(blk + 1, BLK, sem1)

        @pl.when((blk + 1 < NBLK) & (slot == 1))
        def _():
            issue(blk + 1, 0, sem0)

        @pl.when(slot == 0)
        def _():
            drain(blk, 0, sem0)

        @pl.when(slot == 1)
        def _():
            drain(blk, BLK, sem1)

        def grp_step(g, car):
            s1c, s2c, numc, denc, posc, cgc = car
            off0 = soff + g * (GRP * 16)
            p00 = base + blk * BLK + g * (GRP * 16)
            bA = bbuf[pl.ds(off0, 16)]
            bB = bbuf[pl.ds(off0 + (GRP - 1) * 16, 16)]
            gcommon = (bA[0] == cgc) & (bB[15] == cgc)

            cands = []
            ssum = zf
            esum = zf
            for v in range(GRP):
                sv = sbuf[pl.ds(off0 + v * 16, 16)]
                rv = rbuf[pl.ds(off0 + v * 16, 16)]
                uv = ubuf[pl.ds(off0 + v * 16, 16)]
                scv = jnp.maximum(sv, EPS)
                e1v = scv * jnp.exp(rv)
                yv = -_plog(uv)
                posf = (p00 + v * 16 + it).astype(f32)
                ssum = ssum + scv
                esum = esum + e1v
                cands.append((e1v, yv, posf))
            c01 = comb(cands[0], cands[1])
            c23 = comb(cands[2], cands[3])
            c03 = comb(c01, c23)
            gn, gd, gp = comb(c03, cands[4])
            take = gn * denc > numc * gd
            f_num = jnp.where(take, gn, numc)
            f_den = jnp.where(take, gd, denc)
            f_pos = jnp.where(take, gp, posc)
            f_s1 = s1c + ssum
            f_s2 = s2c + esum

            @pl.when(jnp.logical_not(gcommon))
            def _():
                cgv[all16] = jnp.full((16,), cgc, i32)
                as1[all16] = s1c
                as2[all16] = s2c
                anum[all16] = numc
                aden[all16] = denc
                apos[all16] = posc

                def sv_step(v, _):
                    slow_vec(v, off0, p00)
                    return 0
                lax.fori_loop(0, GRP, sv_step, 0)

            s1c = jnp.where(gcommon, f_s1, as1[all16])
            s2c = jnp.where(gcommon, f_s2, as2[all16])
            numc = jnp.where(gcommon, f_num, anum[all16])
            denc = jnp.where(gcommon, f_den, aden[all16])
            posc = jnp.where(gcommon, f_pos, apos[all16])
            cgc = jnp.where(gcommon, cgc, cgv[all16][0])
            return (s1c, s2c, numc, denc, posc, cgc)

        return lax.fori_loop(0, GPB, grp_step, car)

    s1c, s2c, numc, denc, posc, cgc = lax.fori_loop(0, NBLK, block_step, car0)

    cgv[all16] = jnp.full((16,), cgc, i32)
    as1[all16] = s1c
    as2[all16] = s2c
    anum[all16] = numc
    aden[all16] = denc
    apos[all16] = posc

    @pl.when(cgc >= 0)
    def _():
        flush(cgc, jnp.int32(base + CHUNK))

    def cstep(go, _):
        def cinner(j, acc):
            row = packed[pl.ds((go * 16 + j) * 16, 16)]
            return jnp.where(it == j, row[0], acc)
        cntc[pl.ds(go * 16, 16)] = lax.fori_loop(0, 16, cinner, zf)
        return 0
    lax.fori_loop(0, G // 16, cstep, 0)

    pltpu.sync_copy(cntc, cnt_out.at[wid])
    pltpu.sync_copy(packed, pk_out.at[wid])


def _prefix16(v):
    it = _iota()
    for k in (1, 2, 4, 8):
        v = v + jnp.where(it >= k, v[jnp.maximum(it - k, 0)], 0.0)
    return v


def _phase2_body(cnt_hbm, pk_hbm, sr_hbm, us_hbm,
                 act_out, pf_out,
                 cntv, pbuf, totc, ptb,
                 srb, usb, actb, pfb,
                 sem):
    info = plsc.get_sparse_core_info()
    ns = info.num_subcores
    wid = lax.axis_index("c") * ns + lax.axis_index("s")
    gbase = wid * 32
    it = _iota()
    all16 = pl.ds(0, 16)

    pltpu.async_copy(cnt_hbm, cntv, sem)
    for src in range(NW):
        pltpu.async_copy(pk_hbm.at[src, pl.ds(gbase * 16, 512)],
                         pbuf.at[pl.ds(src * 512, 512)], sem)
    pltpu.async_copy(sr_hbm.at[pl.ds(gbase, 32)], srb, sem)
    pltpu.async_copy(us_hbm.at[pl.ds(gbase, 32)], usb, sem)
    pltpu.make_async_copy(cnt_hbm, cntv, sem).wait()
    for src in range(NW):
        pltpu.make_async_copy(pk_hbm.at[src, pl.ds(gbase * 16, 512)],
                              pbuf.at[pl.ds(src * 512, 512)], sem).wait()
    pltpu.make_async_copy(sr_hbm.at[pl.ds(gbase, 32)], srb, sem).wait()
    pltpu.make_async_copy(us_hbm.at[pl.ds(gbase, 32)], usb, sem).wait()

    def tot_step(k, _):
        def src_step(src, acc):
            return acc + cntv[pl.ds(src * G + k * 16, 16)]
        totc[pl.ds(k * 16, 16)] = lax.fori_loop(0, NW, src_step, jnp.zeros((16,), f32))
        return 0
    lax.fori_loop(0, G // 16, tot_step, 0)

    def cum_step(k, cry):
        v = _prefix16(totc[pl.ds(k * 16, 16)]) + cry
        ptb[pl.ds(k * 16, 16)] = v
        return v[15]
    lax.fori_loop(0, G // 16, cum_step, jnp.float32(0))

    sum_lanes = it <= 2

    for h in range(2):
        n16 = jnp.zeros((16,), f32)
        s1m = jnp.zeros((16,), f32)
        s2m = jnp.zeros((16,), f32)
        wm = jnp.zeros((16,), f32)
        am = jnp.zeros((16,), f32)
        e1m = jnp.zeros((16,), f32)
        for j in range(16):
            glocal = h * 16 + j

            def src_step(src, acc, glocal=glocal):
                row = pbuf[pl.ds(src * 512 + glocal * 16, 16)]
                w_p = row[3]
                w_a = acc[3]
                take = (w_p > w_a) | ((w_p == w_a) & (row[4] < acc[4]))
                return jnp.where(sum_lanes, acc + row, jnp.where(take, row, acc))
            acc = lax.fori_loop(0, NW, src_step,
                                jnp.where(it == 4, EF, jnp.where(it == 5, 1.0, 0.0)))
            lane = it == j
            n16 = jnp.where(lane, acc[0], n16)
            s1m = jnp.where(lane, acc[1], s1m)
            s2m = jnp.where(lane, acc[2], s2m)
            wm = jnp.where(lane, acc[3], wm)
            am = jnp.where(lane, acc[4], am)
            e1m = jnp.where(lane, acc[5], e1m)

        has = n16 > 0
        nf = jnp.maximum(n16, 1.0)
        ls1 = -_plog(nf)
        s1safe = jnp.maximum(s1m, 1e-30)
        s2safe = jnp.maximum(s2m, 1e-30)
        lse1 = jnp.where(has, _plogaddexp(_plog(s1safe), ls1), ls1)
        sr16 = srb[pl.ds(h * 16, 16)]
        us16 = usb[pl.ds(h * 16, 16)]
        ls2 = ls1 - lse1 + sr16
        lse2 = jnp.where(has, _plogaddexp(_plog(s2safe) - lse1, ls2), ls2)
        semax = jnp.where(has, _plog(jnp.maximum(wm, 1e-30)) - lse1 - lse2, LARGE_NEG)
        gstop = -_plog(-_plog(us16))
        sstop = ls2 - lse2 + gstop
        choose = has & (semax > sstop)
        argc = jnp.where(has, jnp.clip(am, 0.0, EF - 1.0), 0.0)
        lr = _plog(jnp.maximum(e1m, 1e-30))
        ptr16 = ptb[pl.ds(gbase + h * 16, 16)]
        act = jnp.where(choose, argc, ptr16)
        pf = jnp.where(choose, lr - lse1 - lse2, ls2 - lse2)
        hsl = pl.ds(h * 16, 16)
        actb[hsl] = act.astype(i32)
        pfb[hsl] = pf

    pltpu.sync_copy(actb, act_out.at[pl.ds(gbase, 32)])
    pltpu.sync_copy(pfb, pf_out.at[pl.ds(gbase, 32)])


def kernel(edge_scores, edge_residual, stop_residual, noise_edge_u, noise_stop_u, edge_batch):
    mesh = plsc.VectorSubcoreMesh(core_axis_name="c", subcore_axis_name="s")

    phase1 = pl.kernel(
        _phase1_body,
        out_type=(
            jax.ShapeDtypeStruct((NW, G), f32),
            jax.ShapeDtypeStruct((NW, G * 16), f32),
        ),
        mesh=mesh,
        scratch_types=[
            pltpu.VMEM((2 * BLK,), f32),
            pltpu.VMEM((2 * BLK,), f32),
            pltpu.VMEM((2 * BLK,), f32),
            pltpu.VMEM((2 * BLK,), i32),
            pltpu.VMEM((G * 16,), f32),
            pltpu.VMEM((G,), f32),
            pltpu.VMEM((16,), i32),
            pltpu.VMEM((16,), i32),
            pltpu.VMEM((16,), f32),
            pltpu.VMEM((16,), f32),
            pltpu.VMEM((16,), f32),
            pltpu.VMEM((16,), f32),
            pltpu.VMEM((16,), f32),
            pltpu.VMEM((16,), f32),
            pltpu.SemaphoreType.DMA,
            pltpu.SemaphoreType.DMA,
        ],
    )
    cnt, pk = phase1(edge_scores, edge_residual, noise_edge_u, edge_batch)

    phase2 = pl.kernel(
        _phase2_body,
        out_type=(
            jax.ShapeDtypeStruct((G,), i32),
            jax.ShapeDtypeStruct((G,), f32),
        ),
        mesh=mesh,
        scratch_types=[
            pltpu.VMEM((NW * G,), f32),
            pltpu.VMEM((NW * 512,), f32),
            pltpu.VMEM((G,), f32),
            pltpu.VMEM((G,), f32),
            pltpu.VMEM((32,), f32),
            pltpu.VMEM((32,), f32),
            pltpu.VMEM((32,), i32),
            pltpu.VMEM((32,), f32),
            pltpu.SemaphoreType.DMA,
        ],
    )
    actions, log_pf = phase2(cnt.reshape((NW * G,)), pk,
                             stop_residual, noise_stop_u)
    return actions, log_pf


# speedup vs baseline: 510.9580x; 1.1334x over previous
import jax
import jax.numpy as jnp
from jax import lax
from jax.experimental import pallas as pl
from jax.experimental.pallas import tpu as pltpu
from jax.experimental.pallas import tpu_sc as plsc

E = 1600000
G = 1024
NW = 32
CHUNK = E // NW
BLK = 2000
NBLK = CHUNK // BLK
VPB = BLK // 16
EPS = 1e-6
LN2 = 0.6931471805599453
LARGE_NEG = -1000000000.0
EF = float(E)

f32 = jnp.float32
i32 = jnp.int32


def _iota():
    return lax.iota(i32, 16)


_LC = (2.0086062590962683e-08, 0.9999999403953552, -0.5000073909759521,
       0.33334827423095703, -0.24958817660808563, 0.19907750189304352,
       -0.17360951006412506, 0.16165275871753693, -0.09719803929328918)


def _plog(x):
    bits = lax.bitcast_convert_type(x, i32)
    e = (bits >> 23) - 127
    mbits = (bits & 0x007FFFFF) | 0x3F800000
    m = lax.bitcast_convert_type(mbits, f32)
    big = m > 1.4142135623730951
    m = jnp.where(big, m * 0.5, m)
    e = jnp.where(big, e + 1, e)
    t = m - 1.0
    t2 = t * t
    t4 = t2 * t2
    p = ((_LC[0] + _LC[1] * t) + t2 * (_LC[2] + _LC[3] * t)
         + t4 * ((_LC[4] + _LC[5] * t) + t2 * (_LC[6] + _LC[7] * t) + t4 * _LC[8]))
    return e.astype(f32) * LN2 + p


def _plogaddexp(a, b):
    m = jnp.maximum(a, b)
    d = jnp.minimum(a, b) - m
    return m + _plog(1.0 + jnp.exp(d))


def _bf_sum(v):
    it = _iota()
    for k in (1, 2, 4, 8):
        v = v + v[it ^ k]
    return v


def _bf_maxarg(w, a, e1):
    it = _iota()
    for k in (1, 2, 4, 8):
        wp = w[it ^ k]
        ap = a[it ^ k]
        ep = e1[it ^ k]
        take = (wp > w) | ((wp == w) & (ap < a))
        w = jnp.where(take, wp, w)
        a = jnp.where(take, ap, a)
        e1 = jnp.where(take, ep, e1)
    return w, a, e1


GRP = 5
GPB = VPB // GRP


def _phase1_body(s_hbm, r_hbm, u_hbm, b_hbm,
                 cnt_out, pk_out,
                 sbuf, rbuf, ubuf, bbuf,
                 packed, cntc,
                 cgv, rsv, as1, as2, anum, aden, apos, tmpv,
                 sem0, sem1):
    info = plsc.get_sparse_core_info()
    ns = info.num_subcores
    wid = lax.axis_index("c") * ns + lax.axis_index("s")
    base = wid * CHUNK

    it = _iota()
    l0m = it == 0
    all16 = pl.ds(0, 16)
    zf = jnp.zeros((16,), f32)
    onef = jnp.full((16,), 1.0, f32)
    efv = jnp.full((16,), EF, f32)

    init_row = jnp.where(it == 4, EF, jnp.where(it == 5, 1.0, 0.0))

    def init_step(g, _):
        packed[pl.ds(g * 16, 16)] = init_row
        return 0
    lax.fori_loop(0, G, init_step, 0)

    cgv[all16] = jnp.full((16,), -1, i32)
    rsv[all16] = jnp.zeros((16,), i32)

    hbufs = (s_hbm, r_hbm, u_hbm, b_hbm)
    vbufs = (sbuf, rbuf, ubuf, bbuf)

    def issue(blk, slot_off, sem):
        for h, v in zip(hbufs, vbufs):
            pltpu.async_copy(h.at[pl.ds(base + blk * BLK, BLK)],
                             v.at[pl.ds(slot_off, BLK)], sem)

    def drain(blk, slot_off, sem):
        for h, v in zip(hbufs, vbufs):
            pltpu.make_async_copy(h.at[pl.ds(base + blk * BLK, BLK)],
                                  v.at[pl.ds(slot_off, BLK)], sem).wait()

    def flush(cg, p):
        rs = rsv[all16][0]
        cntf = (p - rs).astype(f32)
        s1 = _bf_sum(as1[all16])
        s2 = _bf_sum(as2[all16])
        nr = anum[all16]
        wlane = nr / aden[all16]
        wr, ar, er = _bf_maxarg(wlane, apos[all16], nr)
        row = jnp.where(it == 0, cntf, 0.0)
        row = jnp.where(it == 1, s1, row)
        row = jnp.where(it == 2, s2, row)
        row = jnp.where(it == 3, wr, row)
        row = jnp.where(it == 4, ar, row)
        row = jnp.where(it == 5, er, row)
        packed[pl.ds(cg * 16, 16)] = row

    def slow_vec(v, off0, p00):
        off = off0 + v * 16
        p0 = p00 + v * 16
        b16 = bbuf[pl.ds(off, 16)]
        s16 = sbuf[pl.ds(off, 16)]
        r16 = rbuf[pl.ds(off, 16)]
        u16 = ubuf[pl.ds(off, 16)]
        scv = jnp.maximum(s16, EPS)
        e1v = scv * jnp.exp(r16)
        yv = -_plog(u16)
        cg = cgv[all16][0]
        common = (b16[0] == cg) & (b16[15] == cg)

        @pl.when(common)
        def _():
            as1[all16] = as1[all16] + scv
            as2[all16] = as2[all16] + e1v
            nr = anum[all16]
            dr = aden[all16]
            up = e1v * dr > nr * yv
            anum[all16] = jnp.where(up, e1v, nr)
            aden[all16] = jnp.where(up, yv, dr)
            posf = (p0 + it).astype(f32)
            apos[all16] = jnp.where(up, posf, apos[all16])

        @pl.when(jnp.logical_not(common))
        def _():
            b16f = b16.astype(f32)

            def elem(i, _):
                isel = (it & 0) + i
                brep = b16f[isel]
                screp = scv[isel]
                e1rep = e1v[isel]
                yrep = yv[isel]
                tmpv[all16] = brep
                b_i = tmpv[all16][0].astype(i32)
                p_i = p0 + i
                cg_i = cgv[all16][0]
                change = b_i != cg_i

                @pl.when(change & (cg_i >= 0))
                def _():
                    flush(cg_i, p_i)

                @pl.when(change)
                def _():
                    cgv[all16] = jnp.full((16,), b_i, i32)
                    rsv[all16] = jnp.full((16,), p_i, i32)
                    as1[all16] = zf
                    as2[all16] = zf
                    anum[all16] = zf
                    aden[all16] = onef
                    apos[all16] = efv

                as1[all16] = as1[all16] + jnp.where(l0m, screp, 0.0)
                as2[all16] = as2[all16] + jnp.where(l0m, e1rep, 0.0)
                nr = anum[all16]
                dr = aden[all16]
                up = l0m & (e1rep * dr > nr * yrep)
                anum[all16] = jnp.where(up, e1rep, nr)
                aden[all16] = jnp.where(up, yrep, dr)
                apos[all16] = jnp.where(up, jnp.full((16,), p_i, i32).astype(f32), apos[all16])
                return 0

            lax.fori_loop(0, 16, elem, 0)

    def comb(a, b):
        an, ad, ap = a
        bn, bd, bp = b
        take = bn * ad > an * bd
        return (jnp.where(take, bn, an), jnp.where(take, bd, ad),
                jnp.where(take, bp, ap))

    issue(0, 0, sem0)
    car0 = (zf, zf, zf, onef, efv, jnp.int32(-1))

    def block_step(blk, car):
        slot = lax.rem(blk, 2)
        soff = slot * BLK

        @pl.when((blk + 1 < NBLK) & (slot == 0))
        def _():
            issue(blk + 1, BLK, sem1)

        @pl.when((blk + 1 < NBLK) & (slot == 1))
        def _():
            issue(blk + 1, 0, sem0)

        @pl.when(slot == 0)
        def _():
            drain(blk, 0, sem0)

        @pl.when(slot == 1)
        def _():
            drain(blk, BLK, sem1)

        def grp_step(g, car):
            s1c, s2c, numc, denc, posc, cgc = car
            off0 = soff + g * (GRP * 16)
            p00 = base + blk * BLK + g * (GRP * 16)
            bA = bbuf[pl.ds(off0, 16)]
            bB = bbuf[pl.ds(off0 + (GRP - 1) * 16, 16)]
            gcommon = (bA[0] == cgc) & (bB[15] == cgc)

            cands = []
            ssum = zf
            esum = zf
            for v in range(GRP):
                sv = sbuf[pl.ds(off0 + v * 16, 16)]
                rv = rbuf[pl.ds(off0 + v * 16, 16)]
                uv = ubuf[pl.ds(off0 + v * 16, 16)]
                scv = jnp.maximum(sv, EPS)
                e1v = scv * jnp.exp(rv)
                yv = -_plog(uv)
                posf = (p00 + v * 16 + it).astype(f32)
                ssum = ssum + scv
                esum = esum + e1v
                cands.append((e1v, yv, posf))
            c01 = comb(cands[0], cands[1])
            c23 = comb(cands[2], cands[3])
            c03 = comb(c01, c23)
            gn, gd, gp = comb(c03, cands[4])
            take = gn * denc > numc * gd
            f_num = jnp.where(take, gn, numc)
            f_den = jnp.where(take, gd, denc)
            f_pos = jnp.where(take, gp, posc)
            f_s1 = s1c + ssum
            f_s2 = s2c + esum

            @pl.when(jnp.logical_not(gcommon))
            def _():
                cgv[all16] = jnp.full((16,), cgc, i32)
                as1[all16] = s1c
                as2[all16] = s2c
                anum[all16] = numc
                aden[all16] = denc
                apos[all16] = posc

                def sv_step(v, _):
                    slow_vec(v, off0, p00)
                    return 0
                lax.fori_loop(0, GRP, sv_step, 0)

            s1c = jnp.where(gcommon, f_s1, as1[all16])
            s2c = jnp.where(gcommon, f_s2, as2[all16])
            numc = jnp.where(gcommon, f_num, anum[all16])
            denc = jnp.where(gcommon, f_den, aden[all16])
            posc = jnp.where(gcommon, f_pos, apos[all16])
            cgc = jnp.where(gcommon, cgc, cgv[all16][0])
            return (s1c, s2c, numc, denc, posc, cgc)

        return lax.fori_loop(0, GPB, grp_step, car)

    s1c, s2c, numc, denc, posc, cgc = lax.fori_loop(0, NBLK, block_step, car0)

    cgv[all16] = jnp.full((16,), cgc, i32)
    as1[all16] = s1c
    as2[all16] = s2c
    anum[all16] = numc
    aden[all16] = denc
    apos[all16] = posc

    @pl.when(cgc >= 0)
    def _():
        flush(cgc, jnp.int32(base + CHUNK))

    def cstep(go, _):
        def cinner(j, acc):
            row = packed[pl.ds((go * 16 + j) * 16, 16)]
            return jnp.where(it == j, row[0], acc)
        cntc[pl.ds(go * 16, 16)] = lax.fori_loop(0, 16, cinner, zf)
        return 0
    lax.fori_loop(0, G // 16, cstep, 0)

    pltpu.sync_copy(cntc, cnt_out.at[wid])
    pltpu.sync_copy(packed, pk_out.at[wid])


def _prefix16(v):
    it = _iota()
    for k in (1, 2, 4, 8):
        v = v + jnp.where(it >= k, v[jnp.maximum(it - k, 0)], 0.0)
    return v


def _phase2_body(cnt_hbm, pk_hbm, sr_hbm, us_hbm,
                 act_out, pf_out,
                 cntv, pbuf, totc, ptb,
                 srb, usb, actb, pfb,
                 sem):
    info = plsc.get_sparse_core_info()
    ns = info.num_subcores
    wid = lax.axis_index("c") * ns + lax.axis_index("s")
    gbase = wid * 32
    it = _iota()
    all16 = pl.ds(0, 16)

    pltpu.async_copy(cnt_hbm, cntv, sem)
    for src in range(NW):
        pltpu.async_copy(pk_hbm.at[src, pl.ds(gbase * 16, 512)],
                         pbuf.at[pl.ds(src * 512, 512)], sem)
    pltpu.async_copy(sr_hbm.at[pl.ds(gbase, 32)], srb, sem)
    pltpu.async_copy(us_hbm.at[pl.ds(gbase, 32)], usb, sem)
    pltpu.make_async_copy(cnt_hbm, cntv, sem).wait()
    for src in range(NW):
        pltpu.make_async_copy(pk_hbm.at[src, pl.ds(gbase * 16, 512)],
                              pbuf.at[pl.ds(src * 512, 512)], sem).wait()
    pltpu.make_async_copy(sr_hbm.at[pl.ds(gbase, 32)], srb, sem).wait()
    pltpu.make_async_copy(us_hbm.at[pl.ds(gbase, 32)], usb, sem).wait()

    def tot_step(k, _):
        vals = [cntv[pl.ds(src * G + k * 16, 16)] for src in range(NW)]
        while len(vals) > 1:
            vals = [vals[a] + vals[a + 1] for a in range(0, len(vals), 2)]
        totc[pl.ds(k * 16, 16)] = vals[0]
        return 0
    lax.fori_loop(0, G // 16, tot_step, 0)

    def cum_step(k, cry):
        v = _prefix16(totc[pl.ds(k * 16, 16)]) + cry
        ptb[pl.ds(k * 16, 16)] = v
        return v[15]
    lax.fori_loop(0, G // 16, cum_step, jnp.float32(0))

    sum_lanes = it <= 2

    def mcomb(a, b):
        take = b[3] > a[3]
        return jnp.where(sum_lanes, a + b, jnp.where(take, b, a))

    def merge_graph(j, car):
        rows = [pbuf[pl.ds(src * 512 + j * 16, 16)] for src in range(NW)]
        while len(rows) > 1:
            rows = [mcomb(rows[a], rows[a + 1]) for a in range(0, len(rows), 2)]
        acc = rows[0]
        lane_lo = it == j
        lane_hi = it == (j - 16)
        out = []
        for q in range(6):
            lo, hi = car[2 * q], car[2 * q + 1]
            out.append(jnp.where(lane_lo, acc[q], lo))
            out.append(jnp.where(lane_hi, acc[q], hi))
        return tuple(out)

    z16 = jnp.zeros((16,), f32)
    mres = lax.fori_loop(0, 32, merge_graph, (z16,) * 12)

    for h in range(2):
        n16 = mres[0 + h]
        s1m = mres[2 + h]
        s2m = mres[4 + h]
        wm = mres[6 + h]
        am = mres[8 + h]
        e1m = mres[10 + h]

        has = n16 > 0
        nf = jnp.maximum(n16, 1.0)
        ls1 = -_plog(nf)
        s1safe = jnp.maximum(s1m, 1e-30)
        s2safe = jnp.maximum(s2m, 1e-30)
        lse1 = jnp.where(has, _plogaddexp(_plog(s1safe), ls1), ls1)
        sr16 = srb[pl.ds(h * 16, 16)]
        us16 = usb[pl.ds(h * 16, 16)]
        ls2 = ls1 - lse1 + sr16
        lse2 = jnp.where(has, _plogaddexp(_plog(s2safe) - lse1, ls2), ls2)
        semax = jnp.where(has, _plog(jnp.maximum(wm, 1e-30)) - lse1 - lse2, LARGE_NEG)
        gstop = -_plog(-_plog(us16))
        sstop = ls2 - lse2 + gstop
        choose = has & (semax > sstop)
        argc = jnp.where(has, jnp.clip(am, 0.0, EF - 1.0), 0.0)
        lr = _plog(jnp.maximum(e1m, 1e-30))
        ptr16 = ptb[pl.ds(gbase + h * 16, 16)]
        act = jnp.where(choose, argc, ptr16)
        pf = jnp.where(choose, lr - lse1 - lse2, ls2 - lse2)
        hsl = pl.ds(h * 16, 16)
        actb[hsl] = act.astype(i32)
        pfb[hsl] = pf

    pltpu.sync_copy(actb, act_out.at[pl.ds(gbase, 32)])
    pltpu.sync_copy(pfb, pf_out.at[pl.ds(gbase, 32)])


def kernel(edge_scores, edge_residual, stop_residual, noise_edge_u, noise_stop_u, edge_batch):
    mesh = plsc.VectorSubcoreMesh(core_axis_name="c", subcore_axis_name="s")

    phase1 = pl.kernel(
        _phase1_body,
        out_type=(
            jax.ShapeDtypeStruct((NW, G), f32),
            jax.ShapeDtypeStruct((NW, G * 16), f32),
        ),
        mesh=mesh,
        scratch_types=[
            pltpu.VMEM((2 * BLK,), f32),
            pltpu.VMEM((2 * BLK,), f32),
            pltpu.VMEM((2 * BLK,), f32),
            pltpu.VMEM((2 * BLK,), i32),
            pltpu.VMEM((G * 16,), f32),
            pltpu.VMEM((G,), f32),
            pltpu.VMEM((16,), i32),
            pltpu.VMEM((16,), i32),
            pltpu.VMEM((16,), f32),
            pltpu.VMEM((16,), f32),
            pltpu.VMEM((16,), f32),
            pltpu.VMEM((16,), f32),
            pltpu.VMEM((16,), f32),
            pltpu.VMEM((16,), f32),
            pltpu.SemaphoreType.DMA,
            pltpu.SemaphoreType.DMA,
        ],
    )
    cnt, pk = phase1(edge_scores, edge_residual, noise_edge_u, edge_batch)

    phase2 = pl.kernel(
        _phase2_body,
        out_type=(
            jax.ShapeDtypeStruct((G,), i32),
            jax.ShapeDtypeStruct((G,), f32),
        ),
        mesh=mesh,
        scratch_types=[
            pltpu.VMEM((NW * G,), f32),
            pltpu.VMEM((NW * 512,), f32),
            pltpu.VMEM((G,), f32),
            pltpu.VMEM((G,), f32),
            pltpu.VMEM((32,), f32),
            pltpu.VMEM((32,), f32),
            pltpu.VMEM((32,), i32),
            pltpu.VMEM((32,), f32),
            pltpu.SemaphoreType.DMA,
        ],
    )
    actions, log_pf = phase2(cnt.reshape((NW * G,)), pk,
                             stop_residual, noise_stop_u)
    return actions, log_pf


# speedup vs baseline: 528.4210x; 1.0342x over previous
import jax
import jax.numpy as jnp
from jax import lax
from jax.experimental import pallas as pl
from jax.experimental.pallas import tpu as pltpu
from jax.experimental.pallas import tpu_sc as plsc

E = 1600000
G = 1024
NW = 32
CHUNK = E // NW
BLK = 2000
NBLK = CHUNK // BLK
VPB = BLK // 16
EPS = 1e-6
LN2 = 0.6931471805599453
LARGE_NEG = -1000000000.0
EF = float(E)

f32 = jnp.float32
i32 = jnp.int32


def _iota():
    return lax.iota(i32, 16)


_LC = (2.0086062590962683e-08, 0.9999999403953552, -0.5000073909759521,
       0.33334827423095703, -0.24958817660808563, 0.19907750189304352,
       -0.17360951006412506, 0.16165275871753693, -0.09719803929328918)


def _plog(x):
    bits = lax.bitcast_convert_type(x, i32)
    e = (bits >> 23) - 127
    mbits = (bits & 0x007FFFFF) | 0x3F800000
    m = lax.bitcast_convert_type(mbits, f32)
    big = m > 1.4142135623730951
    m = jnp.where(big, m * 0.5, m)
    e = jnp.where(big, e + 1, e)
    t = m - 1.0
    t2 = t * t
    t4 = t2 * t2
    p = ((_LC[0] + _LC[1] * t) + t2 * (_LC[2] + _LC[3] * t)
         + t4 * ((_LC[4] + _LC[5] * t) + t2 * (_LC[6] + _LC[7] * t) + t4 * _LC[8]))
    return e.astype(f32) * LN2 + p


def _plogaddexp(a, b):
    m = jnp.maximum(a, b)
    d = jnp.minimum(a, b) - m
    return m + _plog(1.0 + jnp.exp(d))


def _bf_sum(v):
    it = _iota()
    for k in (1, 2, 4, 8):
        v = v + v[it ^ k]
    return v


def _bf_maxarg(w, a, e1):
    it = _iota()
    for k in (1, 2, 4, 8):
        wp = w[it ^ k]
        ap = a[it ^ k]
        ep = e1[it ^ k]
        take = (wp > w) | ((wp == w) & (ap < a))
        w = jnp.where(take, wp, w)
        a = jnp.where(take, ap, a)
        e1 = jnp.where(take, ep, e1)
    return w, a, e1


GRP = 5
GPB = VPB // GRP


def _phase1_body(s_hbm, r_hbm, u_hbm, b_hbm,
                 cnt_out, pk_out,
                 sbuf, rbuf, ubuf, bbuf,
                 packed, cntc,
                 cgv, rsv, as1, as2, anum, aden, apos, tmpv,
                 sem0, sem1):
    info = plsc.get_sparse_core_info()
    ns = info.num_subcores
    wid = lax.axis_index("c") * ns + lax.axis_index("s")
    base = wid * CHUNK

    it = _iota()
    l0m = it == 0
    all16 = pl.ds(0, 16)
    zf = jnp.zeros((16,), f32)
    onef = jnp.full((16,), 1.0, f32)
    efv = jnp.full((16,), EF, f32)

    init_row = jnp.where(it == 4, EF, jnp.where(it == 5, 1.0, 0.0))

    def init_step(g, _):
        packed[pl.ds(g * 16, 16)] = init_row
        return 0
    lax.fori_loop(0, G, init_step, 0)

    cgv[all16] = jnp.full((16,), -1, i32)
    rsv[all16] = jnp.zeros((16,), i32)

    hbufs = (s_hbm, r_hbm, u_hbm, b_hbm)
    vbufs = (sbuf, rbuf, ubuf, bbuf)

    def issue(blk, slot_off, sem):
        for h, v in zip(hbufs, vbufs):
            pltpu.async_copy(h.at[pl.ds(base + blk * BLK, BLK)],
                             v.at[pl.ds(slot_off, BLK)], sem)

    def drain(blk, slot_off, sem):
        for h, v in zip(hbufs, vbufs):
            pltpu.make_async_copy(h.at[pl.ds(base + blk * BLK, BLK)],
                                  v.at[pl.ds(slot_off, BLK)], sem).wait()

    def flush(cg, p):
        rs = rsv[all16][0]
        cntf = (p - rs).astype(f32)
        s1 = _bf_sum(as1[all16])
        s2 = _bf_sum(as2[all16])
        nr = anum[all16]
        wlane = nr / aden[all16]
        wr, ar, er = _bf_maxarg(wlane, apos[all16], nr)
        row = jnp.where(it == 0, cntf, 0.0)
        row = jnp.where(it == 1, s1, row)
        row = jnp.where(it == 2, s2, row)
        row = jnp.where(it == 3, wr, row)
        row = jnp.where(it == 4, ar, row)
        row = jnp.where(it == 5, er, row)
        packed[pl.ds(cg * 16, 16)] = row

    def slow_vec(v, off0, p00):
        off = off0 + v * 16
        p0 = p00 + v * 16
        b16 = bbuf[pl.ds(off, 16)]
        s16 = sbuf[pl.ds(off, 16)]
        r16 = rbuf[pl.ds(off, 16)]
        u16 = ubuf[pl.ds(off, 16)]
        scv = jnp.maximum(s16, EPS)
        e1v = scv * jnp.exp(r16)
        yv = -_plog(u16)
        cg = cgv[all16][0]
        common = (b16[0] == cg) & (b16[15] == cg)

        @pl.when(common)
        def _():
            as1[all16] = as1[all16] + scv
            as2[all16] = as2[all16] + e1v
            nr = anum[all16]
            dr = aden[all16]
            up = e1v * dr > nr * yv
            anum[all16] = jnp.where(up, e1v, nr)
            aden[all16] = jnp.where(up, yv, dr)
            posf = (p0 + it).astype(f32)
            apos[all16] = jnp.where(up, posf, apos[all16])

        @pl.when(jnp.logical_not(common))
        def _():
            b16f = b16.astype(f32)

            def elem(i, _):
                isel = (it & 0) + i
                brep = b16f[isel]
                screp = scv[isel]
                e1rep = e1v[isel]
                yrep = yv[isel]
                tmpv[all16] = brep
                b_i = tmpv[all16][0].astype(i32)
                p_i = p0 + i
                cg_i = cgv[all16][0]
                change = b_i != cg_i

                @pl.when(change & (cg_i >= 0))
                def _():
                    flush(cg_i, p_i)

                @pl.when(change)
                def _():
                    cgv[all16] = jnp.full((16,), b_i, i32)
                    rsv[all16] = jnp.full((16,), p_i, i32)
                    as1[all16] = zf
                    as2[all16] = zf
                    anum[all16] = zf
                    aden[all16] = onef
                    apos[all16] = efv

                as1[all16] = as1[all16] + jnp.where(l0m, screp, 0.0)
                as2[all16] = as2[all16] + jnp.where(l0m, e1rep, 0.0)
                nr = anum[all16]
                dr = aden[all16]
                up = l0m & (e1rep * dr > nr * yrep)
                anum[all16] = jnp.where(up, e1rep, nr)
                aden[all16] = jnp.where(up, yrep, dr)
                apos[all16] = jnp.where(up, jnp.full((16,), p_i, i32).astype(f32), apos[all16])
                return 0

            lax.fori_loop(0, 16, elem, 0)

    def comb(a, b):
        an, ad, ap = a
        bn, bd, bp = b
        take = bn * ad > an * bd
        return (jnp.where(take, bn, an), jnp.where(take, bd, ad),
                jnp.where(take, bp, ap))

    issue(0, 0, sem0)
    car0 = (zf, zf, zf, onef, efv, jnp.int32(-1))

    def block_step(blk, car):
        slot = lax.rem(blk, 2)
        soff = slot * BLK

        @pl.when((blk + 1 < NBLK) & (slot == 0))
        def _():
            issue(blk + 1, BLK, sem1)

        @pl.when((blk + 1 < NBLK) & (slot == 1))
        def _():
            issue(blk + 1, 0, sem0)

        @pl.when(slot == 0)
        def _():
            drain(blk, 0, sem0)

        @pl.when(slot == 1)
        def _():
            drain(blk, BLK, sem1)

        def grp_step_at(g, car):
            s1c, s2c, numc, denc, posc, cgc = car
            off0 = soff + g * (GRP * 16)
            p00 = base + blk * BLK + g * (GRP * 16)
            bA = bbuf[pl.ds(off0, 16)]
            bB = bbuf[pl.ds(off0 + (GRP - 1) * 16, 16)]
            gcommon = (bA[0] == cgc) & (bB[15] == cgc)

            cands = []
            ssum = zf
            esum = zf
            for v in range(GRP):
                sv = sbuf[pl.ds(off0 + v * 16, 16)]
                rv = rbuf[pl.ds(off0 + v * 16, 16)]
                uv = ubuf[pl.ds(off0 + v * 16, 16)]
                scv = jnp.maximum(sv, EPS)
                e1v = scv * jnp.exp(rv)
                yv = -_plog(uv)
                posf = (p00 + v * 16 + it).astype(f32)
                ssum = ssum + scv
                esum = esum + e1v
                cands.append((e1v, yv, posf))
            c01 = comb(cands[0], cands[1])
            c23 = comb(cands[2], cands[3])
            c03 = comb(c01, c23)
            gn, gd, gp = comb(c03, cands[4])
            take = gn * denc > numc * gd
            f_num = jnp.where(take, gn, numc)
            f_den = jnp.where(take, gd, denc)
            f_pos = jnp.where(take, gp, posc)
            f_s1 = s1c + ssum
            f_s2 = s2c + esum

            @pl.when(jnp.logical_not(gcommon))
            def _():
                cgv[all16] = jnp.full((16,), cgc, i32)
                as1[all16] = s1c
                as2[all16] = s2c
                anum[all16] = numc
                aden[all16] = denc
                apos[all16] = posc

                def sv_step(v, _):
                    slow_vec(v, off0, p00)
                    return 0
                lax.fori_loop(0, GRP, sv_step, 0)

            s1c = jnp.where(gcommon, f_s1, as1[all16])
            s2c = jnp.where(gcommon, f_s2, as2[all16])
            numc = jnp.where(gcommon, f_num, anum[all16])
            denc = jnp.where(gcommon, f_den, aden[all16])
            posc = jnp.where(gcommon, f_pos, apos[all16])
            cgc = jnp.where(gcommon, cgc, cgv[all16][0])
            return (s1c, s2c, numc, denc, posc, cgc)

        def grp_pair(d, car):
            car = grp_step_at(2 * d, car)
            return grp_step_at(2 * d + 1, car)

        car = lax.fori_loop(0, GPB // 2, grp_pair, car)
        return grp_step_at(GPB - 1, car)

    s1c, s2c, numc, denc, posc, cgc = lax.fori_loop(0, NBLK, block_step, car0)

    cgv[all16] = jnp.full((16,), cgc, i32)
    as1[all16] = s1c
    as2[all16] = s2c
    anum[all16] = numc
    aden[all16] = denc
    apos[all16] = posc

    @pl.when(cgc >= 0)
    def _():
        flush(cgc, jnp.int32(base + CHUNK))

    def cstep(go, _):
        def cinner(j, acc):
            row = packed[pl.ds((go * 16 + j) * 16, 16)]
            return jnp.where(it == j, row[0], acc)
        cntc[pl.ds(go * 16, 16)] = lax.fori_loop(0, 16, cinner, zf)
        return 0
    lax.fori_loop(0, G // 16, cstep, 0)

    pltpu.sync_copy(cntc, cnt_out.at[wid])
    pltpu.sync_copy(packed, pk_out.at[wid])


def _prefix16(v):
    it = _iota()
    for k in (1, 2, 4, 8):
        v = v + jnp.where(it >= k, v[jnp.maximum(it - k, 0)], 0.0)
    return v


def _phase2_body(cnt_hbm, pk_hbm, sr_hbm, us_hbm,
                 act_out, pf_out,
                 cntv, pbuf, totc, ptb,
                 srb, usb, actb, pfb,
                 sem):
    info = plsc.get_sparse_core_info()
    ns = info.num_subcores
    wid = lax.axis_index("c") * ns + lax.axis_index("s")
    gbase = wid * 32
    it = _iota()
    all16 = pl.ds(0, 16)

    pltpu.async_copy(cnt_hbm, cntv, sem)
    for src in range(NW):
        pltpu.async_copy(pk_hbm.at[src, pl.ds(gbase * 16, 512)],
                         pbuf.at[pl.ds(src * 512, 512)], sem)
    pltpu.async_copy(sr_hbm.at[pl.ds(gbase, 32)], srb, sem)
    pltpu.async_copy(us_hbm.at[pl.ds(gbase, 32)], usb, sem)
    pltpu.make_async_copy(cnt_hbm, cntv, sem).wait()
    for src in range(NW):
        pltpu.make_async_copy(pk_hbm.at[src, pl.ds(gbase * 16, 512)],
                              pbuf.at[pl.ds(src * 512, 512)], sem).wait()
    pltpu.make_async_copy(sr_hbm.at[pl.ds(gbase, 32)], srb, sem).wait()
    pltpu.make_async_copy(us_hbm.at[pl.ds(gbase, 32)], usb, sem).wait()

    def tot_step(k, _):
        vals = [cntv[pl.ds(src * G + k * 16, 16)] for src in range(NW)]
        while len(vals) > 1:
            vals = [vals[a] + vals[a + 1] for a in range(0, len(vals), 2)]
        totc[pl.ds(k * 16, 16)] = vals[0]
        return 0
    lax.fori_loop(0, G // 16, tot_step, 0)

    def cum_step(k, cry):
        v = _prefix16(totc[pl.ds(k * 16, 16)]) + cry
        ptb[pl.ds(k * 16, 16)] = v
        return v[15]
    lax.fori_loop(0, G // 16, cum_step, jnp.float32(0))

    sum_lanes = it <= 2

    def mcomb(a, b):
        take = b[3] > a[3]
        return jnp.where(sum_lanes, a + b, jnp.where(take, b, a))

    def merge_graph(j, car):
        rows = [pbuf[pl.ds(src * 512 + j * 16, 16)] for src in range(NW)]
        while len(rows) > 1:
            rows = [mcomb(rows[a], rows[a + 1]) for a in range(0, len(rows), 2)]
        acc = rows[0]
        lane_lo = it == j
        lane_hi = it == (j - 16)
        out = []
        for q in range(6):
            lo, hi = car[2 * q], car[2 * q + 1]
            out.append(jnp.where(lane_lo, acc[q], lo))
            out.append(jnp.where(lane_hi, acc[q], hi))
        return tuple(out)

    z16 = jnp.zeros((16,), f32)
    mres = lax.fori_loop(0, 32, merge_graph, (z16,) * 12)

    for h in range(2):
        n16 = mres[0 + h]
        s1m = mres[2 + h]
        s2m = mres[4 + h]
        wm = mres[6 + h]
        am = mres[8 + h]
        e1m = mres[10 + h]

        has = n16 > 0
        nf = jnp.maximum(n16, 1.0)
        ls1 = -_plog(nf)
        s1safe = jnp.maximum(s1m, 1e-30)
        s2safe = jnp.maximum(s2m, 1e-30)
        lse1 = jnp.where(has, _plogaddexp(_plog(s1safe), ls1), ls1)
        sr16 = srb[pl.ds(h * 16, 16)]
        us16 = usb[pl.ds(h * 16, 16)]
        ls2 = ls1 - lse1 + sr16
        lse2 = jnp.where(has, _plogaddexp(_plog(s2safe) - lse1, ls2), ls2)
        semax = jnp.where(has, _plog(jnp.maximum(wm, 1e-30)) - lse1 - lse2, LARGE_NEG)
        gstop = -_plog(-_plog(us16))
        sstop = ls2 - lse2 + gstop
        choose = has & (semax > sstop)
        argc = jnp.where(has, jnp.clip(am, 0.0, EF - 1.0), 0.0)
        lr = _plog(jnp.maximum(e1m, 1e-30))
        ptr16 = ptb[pl.ds(gbase + h * 16, 16)]
        act = jnp.where(choose, argc, ptr16)
        pf = jnp.where(choose, lr - lse1 - lse2, ls2 - lse2)
        hsl = pl.ds(h * 16, 16)
        actb[hsl] = act.astype(i32)
        pfb[hsl] = pf

    pltpu.sync_copy(actb, act_out.at[pl.ds(gbase, 32)])
    pltpu.sync_copy(pfb, pf_out.at[pl.ds(gbase, 32)])


def kernel(edge_scores, edge_residual, stop_residual, noise_edge_u, noise_stop_u, edge_batch):
    mesh = plsc.VectorSubcoreMesh(core_axis_name="c", subcore_axis_name="s")

    phase1 = pl.kernel(
        _phase1_body,
        out_type=(
            jax.ShapeDtypeStruct((NW, G), f32),
            jax.ShapeDtypeStruct((NW, G * 16), f32),
        ),
        mesh=mesh,
        scratch_types=[
            pltpu.VMEM((2 * BLK,), f32),
            pltpu.VMEM((2 * BLK,), f32),
            pltpu.VMEM((2 * BLK,), f32),
            pltpu.VMEM((2 * BLK,), i32),
            pltpu.VMEM((G * 16,), f32),
            pltpu.VMEM((G,), f32),
            pltpu.VMEM((16,), i32),
            pltpu.VMEM((16,), i32),
            pltpu.VMEM((16,), f32),
            pltpu.VMEM((16,), f32),
            pltpu.VMEM((16,), f32),
            pltpu.VMEM((16,), f32),
            pltpu.VMEM((16,), f32),
            pltpu.VMEM((16,), f32),
            pltpu.SemaphoreType.DMA,
            pltpu.SemaphoreType.DMA,
        ],
    )
    cnt, pk = phase1(edge_scores, edge_residual, noise_edge_u, edge_batch)

    phase2 = pl.kernel(
        _phase2_body,
        out_type=(
            jax.ShapeDtypeStruct((G,), i32),
            jax.ShapeDtypeStruct((G,), f32),
        ),
        mesh=mesh,
        scratch_types=[
            pltpu.VMEM((NW * G,), f32),
            pltpu.VMEM((NW * 512,), f32),
            pltpu.VMEM((G,), f32),
            pltpu.VMEM((G,), f32),
            pltpu.VMEM((32,), f32),
            pltpu.VMEM((32,), f32),
            pltpu.VMEM((32,), i32),
            pltpu.VMEM((32,), f32),
            pltpu.SemaphoreType.DMA,
        ],
    )
    actions, log_pf = phase2(cnt.reshape((NW * G,)), pk,
                             stop_residual, noise_stop_u)
    return actions, log_pf


# speedup vs baseline: 606.5335x; 1.1478x over previous
import jax
import jax.numpy as jnp
from jax import lax
from jax.experimental import pallas as pl
from jax.experimental.pallas import tpu as pltpu
from jax.experimental.pallas import tpu_sc as plsc

E = 1600000
G = 1024
NW = 32
CHUNK = E // NW
BLK = 2000
NBLK = CHUNK // BLK
VPB = BLK // 16
EPS = 1e-6
LN2 = 0.6931471805599453
LARGE_NEG = -1000000000.0
EF = float(E)

f32 = jnp.float32
i32 = jnp.int32


def _iota():
    return lax.iota(i32, 16)


_LC = (2.0086062590962683e-08, 0.9999999403953552, -0.5000073909759521,
       0.33334827423095703, -0.24958817660808563, 0.19907750189304352,
       -0.17360951006412506, 0.16165275871753693, -0.09719803929328918)


def _plog(x):
    bits = lax.bitcast_convert_type(x, i32)
    e = (bits >> 23) - 127
    mbits = (bits & 0x007FFFFF) | 0x3F800000
    m = lax.bitcast_convert_type(mbits, f32)
    big = m > 1.4142135623730951
    m = jnp.where(big, m * 0.5, m)
    e = jnp.where(big, e + 1, e)
    t = m - 1.0
    t2 = t * t
    t4 = t2 * t2
    p = ((_LC[0] + _LC[1] * t) + t2 * (_LC[2] + _LC[3] * t)
         + t4 * ((_LC[4] + _LC[5] * t) + t2 * (_LC[6] + _LC[7] * t) + t4 * _LC[8]))
    return e.astype(f32) * LN2 + p


def _plogaddexp(a, b):
    m = jnp.maximum(a, b)
    d = jnp.minimum(a, b) - m
    return m + _plog(1.0 + jnp.exp(d))


def _bf_sum(v):
    it = _iota()
    for k in (1, 2, 4, 8):
        v = v + v[it ^ k]
    return v


def _bf_maxarg(w, a, e1):
    it = _iota()
    for k in (1, 2, 4, 8):
        wp = w[it ^ k]
        ap = a[it ^ k]
        ep = e1[it ^ k]
        take = (wp > w) | ((wp == w) & (ap < a))
        w = jnp.where(take, wp, w)
        a = jnp.where(take, ap, a)
        e1 = jnp.where(take, ep, e1)
    return w, a, e1


GRP = 5
GPB = VPB // GRP


def _phase1_body(s_hbm, r_hbm, u_hbm, b_hbm,
                 cnt_out, pk_out,
                 sbuf, rbuf, ubuf, bbuf,
                 packed, cntc,
                 cgv, rsv, as1, as2, anum, aden, apos, tmpv,
                 sem0, sem1):
    info = plsc.get_sparse_core_info()
    ns = info.num_subcores
    wid = lax.axis_index("c") * ns + lax.axis_index("s")
    base = wid * CHUNK

    it = _iota()
    l0m = it == 0
    all16 = pl.ds(0, 16)
    zf = jnp.zeros((16,), f32)
    onef = jnp.full((16,), 1.0, f32)
    efv = jnp.full((16,), EF, f32)

    init_row = jnp.where(it == 4, EF, jnp.where(it == 5, 1.0, 0.0))

    def init_step(g, _):
        packed[pl.ds(g * 16, 16)] = init_row
        return 0
    lax.fori_loop(0, G, init_step, 0)

    cgv[all16] = jnp.full((16,), -1, i32)
    rsv[all16] = jnp.zeros((16,), i32)

    hbufs = (s_hbm, r_hbm, u_hbm, b_hbm)
    vbufs = (sbuf, rbuf, ubuf, bbuf)

    def issue(blk, slot_off, sem):
        for h, v in zip(hbufs, vbufs):
            pltpu.async_copy(h.at[pl.ds(base + blk * BLK, BLK)],
                             v.at[pl.ds(slot_off, BLK)], sem)

    def drain(blk, slot_off, sem):
        for h, v in zip(hbufs, vbufs):
            pltpu.make_async_copy(h.at[pl.ds(base + blk * BLK, BLK)],
                                  v.at[pl.ds(slot_off, BLK)], sem).wait()

    def flush(cg, p):
        rs = rsv[all16][0]
        cntf = (p - rs).astype(f32)
        s1 = _bf_sum(as1[all16])
        s2 = _bf_sum(as2[all16])
        nr = anum[all16]
        wlane = nr / aden[all16]
        wr, ar, er = _bf_maxarg(wlane, apos[all16], nr)
        row = jnp.where(it == 0, cntf, 0.0)
        row = jnp.where(it == 1, s1, row)
        row = jnp.where(it == 2, s2, row)
        row = jnp.where(it == 3, wr, row)
        row = jnp.where(it == 4, ar, row)
        row = jnp.where(it == 5, er, row)
        packed[pl.ds(cg * 16, 16)] = row

    def slow_vec(v, off0, p00):
        off = off0 + v * 16
        p0 = p00 + v * 16
        b16 = bbuf[pl.ds(off, 16)]
        s16 = sbuf[pl.ds(off, 16)]
        r16 = rbuf[pl.ds(off, 16)]
        u16 = ubuf[pl.ds(off, 16)]
        scv = jnp.maximum(s16, EPS)
        e1v = scv * jnp.exp(r16)
        yv = -_plog(u16)
        cg = cgv[all16][0]
        common = (b16[0] == cg) & (b16[15] == cg)

        @pl.when(common)
        def _():
            as1[all16] = as1[all16] + scv
            as2[all16] = as2[all16] + e1v
            nr = anum[all16]
            dr = aden[all16]
            up = e1v * dr > nr * yv
            anum[all16] = jnp.where(up, e1v, nr)
            aden[all16] = jnp.where(up, yv, dr)
            posf = (p0 + it).astype(f32)
            apos[all16] = jnp.where(up, posf, apos[all16])

        def _slow_elems():
            b16f = b16.astype(f32)

            def elem(i, _):
                isel = (it & 0) + i
                brep = b16f[isel]
                screp = scv[isel]
                e1rep = e1v[isel]
                yrep = yv[isel]
                tmpv[all16] = brep
                b_i = tmpv[all16][0].astype(i32)
                p_i = p0 + i
                cg_i = cgv[all16][0]
                change = b_i != cg_i

                @pl.when(change & (cg_i >= 0))
                def _():
                    flush(cg_i, p_i)

                @pl.when(change)
                def _():
                    cgv[all16] = jnp.full((16,), b_i, i32)
                    rsv[all16] = jnp.full((16,), p_i, i32)
                    as1[all16] = zf
                    as2[all16] = zf
                    anum[all16] = zf
                    aden[all16] = onef
                    apos[all16] = efv

                as1[all16] = as1[all16] + jnp.where(l0m, screp, 0.0)
                as2[all16] = as2[all16] + jnp.where(l0m, e1rep, 0.0)
                nr = anum[all16]
                dr = aden[all16]
                up = l0m & (e1rep * dr > nr * yrep)
                anum[all16] = jnp.where(up, e1rep, nr)
                aden[all16] = jnp.where(up, yrep, dr)
                apos[all16] = jnp.where(up, jnp.full((16,), p_i, i32).astype(f32), apos[all16])
                return 0

            lax.fori_loop(0, 16, elem, 0)

        b16f = b16.astype(f32)
        bprev = jnp.where(l0m, jnp.full((16,), cg, i32).astype(f32),
                          b16f[jnp.maximum(it - 1, 0)])
        chg = jnp.where(b16f != bprev, 1.0, 0.0)
        tmpv[all16] = _bf_sum(chg)
        nch = tmpv[all16][0]
        tmpv[all16] = _bf_sum(chg * it.astype(f32))
        k_f = tmpv[all16][0]
        ncommon = jnp.logical_not(common)

        @pl.when(ncommon & (nch == 1.0) & (cg >= 0))
        def _():
            k_i = k_f.astype(i32)
            m1 = it < k_i
            posf = (p0 + it).astype(f32)
            as1[all16] = as1[all16] + jnp.where(m1, scv, 0.0)
            as2[all16] = as2[all16] + jnp.where(m1, e1v, 0.0)
            nr = anum[all16]
            dr = aden[all16]
            up = m1 & (e1v * dr > nr * yv)
            anum[all16] = jnp.where(up, e1v, nr)
            aden[all16] = jnp.where(up, yv, dr)
            apos[all16] = jnp.where(up, posf, apos[all16])
            flush(cg, p0 + k_i)

        @pl.when(ncommon & (nch == 1.0))
        def _():
            k_i = k_f.astype(i32)
            m2 = it >= k_i
            posf = (p0 + it).astype(f32)
            g2 = b16[15]
            cgv[all16] = jnp.full((16,), g2, i32)
            rsv[all16] = jnp.full((16,), p0 + k_i, i32)
            as1[all16] = jnp.where(m2, scv, 0.0)
            as2[all16] = jnp.where(m2, e1v, 0.0)
            anum[all16] = jnp.where(m2, e1v, 0.0)
            aden[all16] = jnp.where(m2, yv, 1.0)
            apos[all16] = jnp.where(m2, posf, EF)

        @pl.when(ncommon & jnp.logical_not(nch == 1.0))
        def _():
            _slow_elems()

    def comb(a, b):
        an, ad, ap = a
        bn, bd, bp = b
        take = bn * ad > an * bd
        return (jnp.where(take, bn, an), jnp.where(take, bd, ad),
                jnp.where(take, bp, ap))

    issue(0, 0, sem0)
    car0 = (zf, zf, zf, onef, efv, jnp.int32(-1))

    def block_step(blk, car):
        slot = lax.rem(blk, 2)
        soff = slot * BLK

        @pl.when((blk + 1 < NBLK) & (slot == 0))
        def _():
            issue(blk + 1, BLK, sem1)

        @pl.when((blk + 1 < NBLK) & (slot == 1))
        def _():
            issue(blk + 1, 0, sem0)

        @pl.when(slot == 0)
        def _():
            drain(blk, 0, sem0)

        @pl.when(slot == 1)
        def _():
            drain(blk, BLK, sem1)

        def grp_step_at(g, car):
            s1c, s2c, numc, denc, posc, cgc = car
            off0 = soff + g * (GRP * 16)
            p00 = base + blk * BLK + g * (GRP * 16)
            bA = bbuf[pl.ds(off0, 16)]
            bB = bbuf[pl.ds(off0 + (GRP - 1) * 16, 16)]
            gcommon = (bA[0] == cgc) & (bB[15] == cgc)

            cands = []
            ssum = zf
            esum = zf
            for v in range(GRP):
                sv = sbuf[pl.ds(off0 + v * 16, 16)]
                rv = rbuf[pl.ds(off0 + v * 16, 16)]
                uv = ubuf[pl.ds(off0 + v * 16, 16)]
                scv = jnp.maximum(sv, EPS)
                e1v = scv * jnp.exp(rv)
                yv = -_plog(uv)
                posf = (p00 + v * 16 + it).astype(f32)
                ssum = ssum + scv
                esum = esum + e1v
                cands.append((e1v, yv, posf))
            c01 = comb(cands[0], cands[1])
            c23 = comb(cands[2], cands[3])
            c03 = comb(c01, c23)
            gn, gd, gp = comb(c03, cands[4])
            take = gn * denc > numc * gd
            f_num = jnp.where(take, gn, numc)
            f_den = jnp.where(take, gd, denc)
            f_pos = jnp.where(take, gp, posc)
            f_s1 = s1c + ssum
            f_s2 = s2c + esum

            @pl.when(jnp.logical_not(gcommon))
            def _():
                cgv[all16] = jnp.full((16,), cgc, i32)
                as1[all16] = s1c
                as2[all16] = s2c
                anum[all16] = numc
                aden[all16] = denc
                apos[all16] = posc

                def sv_step(v, _):
                    slow_vec(v, off0, p00)
                    return 0
                lax.fori_loop(0, GRP, sv_step, 0)

            s1c = jnp.where(gcommon, f_s1, as1[all16])
            s2c = jnp.where(gcommon, f_s2, as2[all16])
            numc = jnp.where(gcommon, f_num, anum[all16])
            denc = jnp.where(gcommon, f_den, aden[all16])
            posc = jnp.where(gcommon, f_pos, apos[all16])
            cgc = jnp.where(gcommon, cgc, cgv[all16][0])
            return (s1c, s2c, numc, denc, posc, cgc)

        def grp_pair(d, car):
            car = grp_step_at(2 * d, car)
            return grp_step_at(2 * d + 1, car)

        car = lax.fori_loop(0, GPB // 2, grp_pair, car)
        return grp_step_at(GPB - 1, car)

    s1c, s2c, numc, denc, posc, cgc = lax.fori_loop(0, NBLK, block_step, car0)

    cgv[all16] = jnp.full((16,), cgc, i32)
    as1[all16] = s1c
    as2[all16] = s2c
    anum[all16] = numc
    aden[all16] = denc
    apos[all16] = posc

    @pl.when(cgc >= 0)
    def _():
        flush(cgc, jnp.int32(base + CHUNK))

    def cstep(go, _):
        def cinner(j, acc):
            row = packed[pl.ds((go * 16 + j) * 16, 16)]
            return jnp.where(it == j, row[0], acc)
        cntc[pl.ds(go * 16, 16)] = lax.fori_loop(0, 16, cinner, zf)
        return 0
    lax.fori_loop(0, G // 16, cstep, 0)

    pltpu.sync_copy(cntc, cnt_out.at[wid])
    pltpu.sync_copy(packed, pk_out.at[wid])


def _prefix16(v):
    it = _iota()
    for k in (1, 2, 4, 8):
        v = v + jnp.where(it >= k, v[jnp.maximum(it - k, 0)], 0.0)
    return v


def _phase2_body(cnt_hbm, pk_hbm, sr_hbm, us_hbm,
                 act_out, pf_out,
                 cntv, pbuf, totc, ptb,
                 srb, usb, actb, pfb,
                 sem):
    info = plsc.get_sparse_core_info()
    ns = info.num_subcores
    wid = lax.axis_index("c") * ns + lax.axis_index("s")
    gbase = wid * 32
    it = _iota()
    all16 = pl.ds(0, 16)

    pltpu.async_copy(cnt_hbm, cntv, sem)
    for src in range(NW):
        pltpu.async_copy(pk_hbm.at[src, pl.ds(gbase * 16, 512)],
                         pbuf.at[pl.ds(src * 512, 512)], sem)
    pltpu.async_copy(sr_hbm.at[pl.ds(gbase, 32)], srb, sem)
    pltpu.async_copy(us_hbm.at[pl.ds(gbase, 32)], usb, sem)
    pltpu.make_async_copy(cnt_hbm, cntv, sem).wait()
    for src in range(NW):
        pltpu.make_async_copy(pk_hbm.at[src, pl.ds(gbase * 16, 512)],
                              pbuf.at[pl.ds(src * 512, 512)], sem).wait()
    pltpu.make_async_copy(sr_hbm.at[pl.ds(gbase, 32)], srb, sem).wait()
    pltpu.make_async_copy(us_hbm.at[pl.ds(gbase, 32)], usb, sem).wait()

    def tot_step(k, _):
        vals = [cntv[pl.ds(src * G + k * 16, 16)] for src in range(NW)]
        while len(vals) > 1:
            vals = [vals[a] + vals[a + 1] for a in range(0, len(vals), 2)]
        totc[pl.ds(k * 16, 16)] = vals[0]
        return 0
    lax.fori_loop(0, G // 16, tot_step, 0)

    def cum_step(k, cry):
        v = _prefix16(totc[pl.ds(k * 16, 16)]) + cry
        ptb[pl.ds(k * 16, 16)] = v
        return v[15]
    lax.fori_loop(0, G // 16, cum_step, jnp.float32(0))

    sum_lanes = it <= 2

    def mcomb(a, b):
        take = b[3] > a[3]
        return jnp.where(sum_lanes, a + b, jnp.where(take, b, a))

    def merge_graph(j, car):
        rows = [pbuf[pl.ds(src * 512 + j * 16, 16)] for src in range(NW)]
        while len(rows) > 1:
            rows = [mcomb(rows[a], rows[a + 1]) for a in range(0, len(rows), 2)]
        acc = rows[0]
        lane_lo = it == j
        lane_hi = it == (j - 16)
        out = []
        for q in range(6):
            lo, hi = car[2 * q], car[2 * q + 1]
            out.append(jnp.where(lane_lo, acc[q], lo))
            out.append(jnp.where(lane_hi, acc[q], hi))
        return tuple(out)

    z16 = jnp.zeros((16,), f32)
    mres = lax.fori_loop(0, 32, merge_graph, (z16,) * 12)

    for h in range(2):
        n16 = mres[0 + h]
        s1m = mres[2 + h]
        s2m = mres[4 + h]
        wm = mres[6 + h]
        am = mres[8 + h]
        e1m = mres[10 + h]

        has = n16 > 0
        nf = jnp.maximum(n16, 1.0)
        ls1 = -_plog(nf)
        s1safe = jnp.maximum(s1m, 1e-30)
        s2safe = jnp.maximum(s2m, 1e-30)
        lse1 = jnp.where(has, _plogaddexp(_plog(s1safe), ls1), ls1)
        sr16 = srb[pl.ds(h * 16, 16)]
        us16 = usb[pl.ds(h * 16, 16)]
        ls2 = ls1 - lse1 + sr16
        lse2 = jnp.where(has, _plogaddexp(_plog(s2safe) - lse1, ls2), ls2)
        semax = jnp.where(has, _plog(jnp.maximum(wm, 1e-30)) - lse1 - lse2, LARGE_NEG)
        gstop = -_plog(-_plog(us16))
        sstop = ls2 - lse2 + gstop
        choose = has & (semax > sstop)
        argc = jnp.where(has, jnp.clip(am, 0.0, EF - 1.0), 0.0)
        lr = _plog(jnp.maximum(e1m, 1e-30))
        ptr16 = ptb[pl.ds(gbase + h * 16, 16)]
        act = jnp.where(choose, argc, ptr16)
        pf = jnp.where(choose, lr - lse1 - lse2, ls2 - lse2)
        hsl = pl.ds(h * 16, 16)
        actb[hsl] = act.astype(i32)
        pfb[hsl] = pf

    pltpu.sync_copy(actb, act_out.at[pl.ds(gbase, 32)])
    pltpu.sync_copy(pfb, pf_out.at[pl.ds(gbase, 32)])


def kernel(edge_scores, edge_residual, stop_residual, noise_edge_u, noise_stop_u, edge_batch):
    mesh = plsc.VectorSubcoreMesh(core_axis_name="c", subcore_axis_name="s")

    phase1 = pl.kernel(
        _phase1_body,
        out_type=(
            jax.ShapeDtypeStruct((NW, G), f32),
            jax.ShapeDtypeStruct((NW, G * 16), f32),
        ),
        mesh=mesh,
        scratch_types=[
            pltpu.VMEM((2 * BLK,), f32),
            pltpu.VMEM((2 * BLK,), f32),
            pltpu.VMEM((2 * BLK,), f32),
            pltpu.VMEM((2 * BLK,), i32),
            pltpu.VMEM((G * 16,), f32),
            pltpu.VMEM((G,), f32),
            pltpu.VMEM((16,), i32),
            pltpu.VMEM((16,), i32),
            pltpu.VMEM((16,), f32),
            pltpu.VMEM((16,), f32),
            pltpu.VMEM((16,), f32),
            pltpu.VMEM((16,), f32),
            pltpu.VMEM((16,), f32),
            pltpu.VMEM((16,), f32),
            pltpu.SemaphoreType.DMA,
            pltpu.SemaphoreType.DMA,
        ],
    )
    cnt, pk = phase1(edge_scores, edge_residual, noise_edge_u, edge_batch)

    phase2 = pl.kernel(
        _phase2_body,
        out_type=(
            jax.ShapeDtypeStruct((G,), i32),
            jax.ShapeDtypeStruct((G,), f32),
        ),
        mesh=mesh,
        scratch_types=[
            pltpu.VMEM((NW * G,), f32),
            pltpu.VMEM((NW * 512,), f32),
            pltpu.VMEM((G,), f32),
            pltpu.VMEM((G,), f32),
            pltpu.VMEM((32,), f32),
            pltpu.VMEM((32,), f32),
            pltpu.VMEM((32,), i32),
            pltpu.VMEM((32,), f32),
            pltpu.SemaphoreType.DMA,
        ],
    )
    actions, log_pf = phase2(cnt.reshape((NW * G,)), pk,
                             stop_residual, noise_stop_u)
    return actions, log_pf
